# baseline (speedup 1.0000x reference)
import functools

import jax
import jax.numpy as jnp
from jax import lax
from jax.experimental import pallas as pl
from jax.experimental.pallas import tpu as pltpu
from jax.experimental.pallas import tpu_sc as plsc

N_NODES = 100000
N_EDGES = 6400000
N_PAD = 100096
E_PAD = 6422528
PAD_E = E_PAD - N_EDGES
EROWS = E_PAD // 128
CHUNK = 25600
ACC2 = CHUNK + 8
CAP = 1280
R = 2000
GRID = N_NODES // R

_mesh = plsc.VectorSubcoreMesh(core_axis_name="c", subcore_axis_name="s")
_f32 = jnp.float32
_i32 = jnp.int32


@functools.partial(
    pl.kernel,
    out_type=jax.ShapeDtypeStruct((32, N_PAD), _f32),
    mesh=_mesh,
    compiler_params=pltpu.CompilerParams(needs_layout_passes=False),
    scratch_types=[
        pltpu.VMEM((N_PAD,), _f32),
        pltpu.VMEM((32, 128), _i32),
    ],
)
def _deg_sc(dst_hbm, out_hbm, acc, blk):
    c = lax.axis_index("c")
    s = lax.axis_index("s")
    wid = c * 16 + s
    zeros = jnp.zeros((16,), _f32)
    ones = jnp.ones((16,), _f32)

    def _zero(i, _):
        acc[pl.ds(i * 16, 16)] = zeros
        return _
    lax.fori_loop(0, N_PAD // 16, _zero, None)

    rows_per_tile = EROWS // 32
    base = wid * rows_per_tile

    def _block(b, _):
        pltpu.sync_copy(dst_hbm.at[pl.ds(base + b * 32, 32), :], blk)

        def _hist(j, _):
            g = j // 8
            k = j % 8
            idx = blk[g, pl.ds(k * 16, 16)]
            plsc.addupdate_scatter(acc, [idx], ones)
            return _
        lax.fori_loop(0, 32 * 8, _hist, None)
        return _
    lax.fori_loop(0, rows_per_tile // 32, _block, None)

    pltpu.sync_copy(acc, out_hbm.at[wid])


@functools.partial(
    pl.kernel,
    out_type=jax.ShapeDtypeStruct((2, N_PAD, 16), _f32),
    mesh=_mesh,
    compiler_params=pltpu.CompilerParams(
        needs_layout_passes=False, use_tc_tiling_on_sc=False),
    scratch_types=[
        pltpu.VMEM_SHARED((N_PAD, 16), _f32),
        pltpu.VMEM((8, 128), _i32),
        pltpu.VMEM((8, 128), _i32),
        pltpu.VMEM((1024, 16), _f32),
        pltpu.SemaphoreType.DMA,
        pltpu.SemaphoreType.DMA,
    ],
)
def _seg1_sc(src_hbm, dst_hbm, u16_hbm, out_hbm, acc, sblk, dblk, rows, sem,
             sem2):
    c = lax.axis_index("c")
    s = lax.axis_index("s")
    zeros = jnp.zeros((16,), _f32)

    def _zb(i, _):
        rows[i, :] = zeros
        return _
    lax.fori_loop(0, 782, _zb, None)

    zr = s * (N_PAD // 16)

    def _za(k, _):
        pltpu.sync_copy(
            rows.at[pl.ds(0, 782), :], acc.at[pl.ds(zr + k * 782, 782), :])
        return _
    lax.fori_loop(0, 8, _za, None)
    plsc.subcore_barrier()

    rows_per_tile = EROWS // 32
    base = c * (EROWS // 2) + s * rows_per_tile

    def _block(b, _):
        r0 = base + b * 8
        pltpu.sync_copy(src_hbm.at[pl.ds(r0, 8), :], sblk)
        pltpu.sync_copy(dst_hbm.at[pl.ds(r0, 8), :], dblk)
        hs = [
            pltpu.async_copy(
                u16_hbm.at[sblk.at[g]], rows.at[pl.ds(g * 128, 128), :], sem)
            for g in range(8)
        ]
        ws = []
        for g in range(8):
            hs[g].wait()
            ws.append(pltpu.async_copy(
                rows.at[pl.ds(g * 128, 128), :], acc.at[dblk.at[g]], sem2,
                add=True))
        for w in ws:
            w.wait()
        return _
    lax.fori_loop(0, rows_per_tile // 8, _block, None)
    plsc.subcore_barrier()

    pltpu.sync_copy(
        acc.at[pl.ds(zr, N_PAD // 16), :],
        out_hbm.at[c, pl.ds(zr, N_PAD // 16), :])


@functools.partial(
    pl.kernel,
    out_type=jax.ShapeDtypeStruct((4 * CHUNK, 64), _f32),
    mesh=_mesh,
    compiler_params=pltpu.CompilerParams(
        needs_layout_passes=False, use_tc_tiling_on_sc=False),
    scratch_types=[
        pltpu.VMEM_SHARED((ACC2, 64), _f32),
        pltpu.VMEM((8, 128), _i32),
        pltpu.VMEM((8, 128), _i32),
        pltpu.VMEM((CAP,), _i32),
        pltpu.VMEM((CAP,), _i32),
        pltpu.VMEM((2, 128), _i32),
        pltpu.VMEM((256, 64), _f32),
        pltpu.SemaphoreType.DMA,
        pltpu.SemaphoreType.DMA,
        pltpu.SemaphoreType.DMA,
    ],
)
def _seg2_sc(src_hbm, dst_hbm, u2_hbm, out_hbm,
             acc, sblk, dblk, csrc, cdst, didx, rowbuf, sem, sc0, sc1):
    c = lax.axis_index("c")
    s = lax.axis_index("s")
    zeros = jnp.zeros((16,), _f32)

    rows_per_tile = EROWS // 16
    base = s * rows_per_tile

    for p in range(2):
        lo = (c + 2 * p) * CHUNK

        def _zb(i, _):
            g = i // 4
            k = i % 4
            rowbuf[g, pl.ds(k * 16, 16)] = zeros
            return _
        lax.fori_loop(0, 64 * 4, _zb, None)

        def _za(k, _):
            pltpu.sync_copy(
                rowbuf.at[pl.ds(0, 64), :],
                acc.at[pl.ds(s * (CHUNK // 16) + k * 64, 64), :])
            return _
        lax.fori_loop(0, 25, _za, None)
        plsc.subcore_barrier()

        def _block(b, cnt):
            r0 = base + b * 8
            pltpu.sync_copy(src_hbm.at[pl.ds(r0, 8), :], sblk)
            pltpu.sync_copy(dst_hbm.at[pl.ds(r0, 8), :], dblk)

            def _comp(j, cnt):
                g = j // 8
                k = j % 8
                d = dblk[g, pl.ds(k * 16, 16)]
                sv = sblk[g, pl.ds(k * 16, 16)]
                inm = (d >= lo) & (d < lo + CHUNK)
                plsc.store_compressed(csrc.at[pl.ds(cnt, 16)], sv, mask=inm)
                plsc.store_compressed(cdst.at[pl.ds(cnt, 16)], d - lo, mask=inm)
                pcv = plsc.all_reduce_population_count(inm)
                pc = pcv if pcv.ndim == 0 else pcv[0]
                return cnt + pc
            cnt = lax.fori_loop(0, 64, _comp, cnt)

            nfull = lax.shift_right_logical(cnt, 7)

            def _row(r, q, scq):
                off = pl.multiple_of(r * 128, 128)
                half = rowbuf.at[pl.ds(q * 128, 128), :]

                @pl.when(r >= 2)
                def _():
                    pltpu.make_async_copy(
                        u2_hbm.at[pl.ds(0, 128), :], half, scq).wait()
                hg = pltpu.async_copy(
                    u2_hbm.at[csrc.at[pl.ds(off, 128)]], half, sem)

                def _fd(j, _):
                    didx[q, pl.ds(j * 16, 16)] = cdst[pl.ds(off + j * 16, 16)]
                    return _
                lax.fori_loop(0, 8, _fd, None)
                hg.wait()
                pltpu.async_copy(half, acc.at[didx.at[q]], scq, add=True)

            def _drain(r, _):
                @pl.when((r & 1) == 0)
                def _():
                    _row(r, 0, sc0)

                @pl.when((r & 1) == 1)
                def _():
                    _row(r, 1, sc1)
                return _
            lax.fori_loop(0, nfull, _drain, None)

            @pl.when(nfull >= 1)
            def _():
                pltpu.make_async_copy(
                    u2_hbm.at[pl.ds(0, 128), :],
                    rowbuf.at[pl.ds(0, 128), :], sc0).wait()

            @pl.when(nfull >= 2)
            def _():
                pltpu.make_async_copy(
                    u2_hbm.at[pl.ds(0, 128), :],
                    rowbuf.at[pl.ds(128, 128), :], sc1).wait()

            @pl.when(nfull > 0)
            def _mv():
                def _mvv(j, _):
                    csrc[pl.ds(j * 16, 16)] = csrc[pl.ds(nfull * 128 + j * 16, 16)]
                    cdst[pl.ds(j * 16, 16)] = cdst[pl.ds(nfull * 128 + j * 16, 16)]
                    return _
                lax.fori_loop(0, 8, _mvv, None)
            return cnt & 127
        cnt = lax.fori_loop(0, rows_per_tile // 8, _block, jnp.int32(0))

        @pl.when(cnt > 0)
        def _flush():
            def _pad(j, _):
                csrc[pl.ds(cnt + j * 16, 16)] = jnp.zeros((16,), _i32)
                cdst[pl.ds(cnt + j * 16, 16)] = jnp.full((16,), CHUNK, _i32)
                return _
            lax.fori_loop(0, 8, _pad, None)
            hf = pltpu.async_copy(
                u2_hbm.at[csrc.at[pl.ds(0, 128)]],
                rowbuf.at[pl.ds(0, 128), :], sem)

            def _fd(j, _):
                didx[0, pl.ds(j * 16, 16)] = cdst[pl.ds(j * 16, 16)]
                return _
            lax.fori_loop(0, 8, _fd, None)
            hf.wait()
            pltpu.sync_copy(
                rowbuf.at[pl.ds(0, 128), :], acc.at[didx.at[0]], add=True)

        plsc.subcore_barrier()
        pltpu.sync_copy(
            acc.at[pl.ds(s * (CHUNK // 16), CHUNK // 16), :],
            out_hbm.at[pl.ds(lo + s * (CHUNK // 16), CHUNK // 16), :])
        plsc.subcore_barrier()


def _stage_a0(dp_ref, deg_ref):
    deg_ref[...] = jnp.sum(dp_ref[...], axis=0)[:, None]


def _stage_a(deg_ref, x_ref, dis_ref, u16_ref):
    deg = deg_ref[...][:, 0] + 1.0
    dis = lax.rsqrt(deg)
    dis_ref[...] = dis[:, None]
    u16_ref[...] = jnp.concatenate(
        [dis[:, None] * x_ref[...], jnp.zeros((R, 14), _f32)], axis=1)


def _stage_b(t1_ref, u16_ref, dis_ref, w1_ref, b1_ref, w2_ref, u2_ref):
    t1 = t1_ref[0] + t1_ref[1]
    dis = dis_ref[...]
    s1 = dis * (t1[:, :2] + u16_ref[..., :2])
    h1 = jnp.maximum(
        jnp.dot(s1, w1_ref[...], preferred_element_type=_f32) + b1_ref[...],
        0.0)
    z = jnp.dot(h1, w2_ref[...], preferred_element_type=_f32)
    u2_ref[...] = dis * z


def _stage_c(t2_ref, u2_ref, dis_ref, b2_ref, wl_ref, bl_ref, out_ref, acc):
    i = pl.program_id(0)
    h2 = jnp.maximum(
        dis_ref[...] * (t2_ref[...] + u2_ref[...]) + b2_ref[...], 0.0)
    part = jnp.sum(h2, axis=0, keepdims=True)

    @pl.when(i == 0)
    def _():
        acc[...] = part

    @pl.when(i > 0)
    def _():
        acc[...] = acc[...] + part

    @pl.when(i == GRID - 1)
    def _():
        pooled = acc[...] / float(N_NODES)
        out_ref[...] = (
            jnp.dot(pooled, wl_ref[...], preferred_element_type=_f32)
            + bl_ref[...])


def kernel(x, edge_index, W1, b1, W2, b2, Wl, bl):
    ei = edge_index.astype(_i32)
    src = jnp.concatenate([ei[0], jnp.zeros((PAD_E,), _i32)])
    dst = jnp.concatenate([ei[1], jnp.full((PAD_E,), N_NODES, _i32)])
    src2d = src.reshape(EROWS, 128)
    dst2d = dst.reshape(EROWS, 128)

    degpart = _deg_sc(dst2d)

    deg2d = pl.pallas_call(
        _stage_a0,
        grid=(17,),
        in_specs=[pl.BlockSpec((32, 5888), lambda i: (0, i))],
        out_specs=pl.BlockSpec((5888, 1), lambda i: (i, 0)),
        out_shape=jax.ShapeDtypeStruct((N_PAD, 1), _f32),
    )(degpart)

    dis, u16 = pl.pallas_call(
        _stage_a,
        grid=(GRID,),
        in_specs=[
            pl.BlockSpec((R, 1), lambda i: (i, 0)),
            pl.BlockSpec((R, 2), lambda i: (i, 0)),
        ],
        out_specs=[
            pl.BlockSpec((R, 1), lambda i: (i, 0)),
            pl.BlockSpec((R, 16), lambda i: (i, 0)),
        ],
        out_shape=[
            jax.ShapeDtypeStruct((N_NODES, 1), _f32),
            jax.ShapeDtypeStruct((N_NODES, 16), _f32),
        ],
    )(deg2d, x)

    t1part = _seg1_sc(src2d, dst2d, u16)

    u2 = pl.pallas_call(
        _stage_b,
        grid=(GRID,),
        in_specs=[
            pl.BlockSpec((2, R, 16), lambda i: (0, i, 0)),
            pl.BlockSpec((R, 16), lambda i: (i, 0)),
            pl.BlockSpec((R, 1), lambda i: (i, 0)),
            pl.BlockSpec((2, 64), lambda i: (0, 0)),
            pl.BlockSpec((1, 64), lambda i: (0, 0)),
            pl.BlockSpec((64, 64), lambda i: (0, 0)),
        ],
        out_specs=pl.BlockSpec((R, 64), lambda i: (i, 0)),
        out_shape=jax.ShapeDtypeStruct((N_NODES, 64), _f32),
    )(t1part, u16, dis, W1, b1.reshape(1, 64), W2)

    t2 = _seg2_sc(src2d, dst2d, u2)

    out = pl.pallas_call(
        _stage_c,
        grid=(GRID,),
        in_specs=[
            pl.BlockSpec((R, 64), lambda i: (i, 0)),
            pl.BlockSpec((R, 64), lambda i: (i, 0)),
            pl.BlockSpec((R, 1), lambda i: (i, 0)),
            pl.BlockSpec((1, 64), lambda i: (0, 0)),
            pl.BlockSpec((64, 1), lambda i: (0, 0)),
            pl.BlockSpec((1, 1), lambda i: (0, 0)),
        ],
        out_specs=pl.BlockSpec((1, 1), lambda i: (0, 0)),
        out_shape=jax.ShapeDtypeStruct((1, 1), _f32),
        scratch_shapes=[pltpu.VMEM((1, 64), _f32)],
    )(t2, u2, dis, b2.reshape(1, 64), Wl, bl.reshape(1, 1))

    return out.reshape(1)


# speedup vs baseline: 1.0237x; 1.0237x over previous
import functools

import jax
import jax.numpy as jnp
from jax import lax
from jax.experimental import pallas as pl
from jax.experimental.pallas import tpu as pltpu
from jax.experimental.pallas import tpu_sc as plsc

N_NODES = 100000
N_EDGES = 6400000
N_PAD = 100096
E_PAD = 6422528
PAD_E = E_PAD - N_EDGES
EROWS = E_PAD // 128
CHUNK = 25600
ACC2 = CHUNK + 8
CAP = 1280
R = 2000
GRID = N_NODES // R

_mesh = plsc.VectorSubcoreMesh(core_axis_name="c", subcore_axis_name="s")
_f32 = jnp.float32
_i32 = jnp.int32


@functools.partial(
    pl.kernel,
    out_type=jax.ShapeDtypeStruct((32, N_PAD), _f32),
    mesh=_mesh,
    compiler_params=pltpu.CompilerParams(needs_layout_passes=False),
    scratch_types=[
        pltpu.VMEM((N_PAD,), _f32),
        pltpu.VMEM((32, 128), _i32),
    ],
)
def _deg_sc(dst_hbm, out_hbm, acc, blk):
    c = lax.axis_index("c")
    s = lax.axis_index("s")
    wid = c * 16 + s
    zeros = jnp.zeros((16,), _f32)
    ones = jnp.ones((16,), _f32)

    def _zero(i, _):
        acc[pl.ds(i * 16, 16)] = zeros
        return _
    lax.fori_loop(0, N_PAD // 16, _zero, None)

    rows_per_tile = EROWS // 32
    base = wid * rows_per_tile

    def _block(b, _):
        pltpu.sync_copy(dst_hbm.at[pl.ds(base + b * 32, 32), :], blk)

        def _hist(j, _):
            g = j // 8
            k = j % 8
            idx = blk[g, pl.ds(k * 16, 16)]
            plsc.addupdate_scatter(acc, [idx], ones)
            return _
        lax.fori_loop(0, 32 * 8, _hist, None)
        return _
    lax.fori_loop(0, rows_per_tile // 32, _block, None)

    pltpu.sync_copy(acc, out_hbm.at[wid])


@functools.partial(
    pl.kernel,
    out_type=jax.ShapeDtypeStruct((2, N_PAD, 16), _f32),
    mesh=_mesh,
    compiler_params=pltpu.CompilerParams(
        needs_layout_passes=False, use_tc_tiling_on_sc=False),
    scratch_types=[
        pltpu.VMEM_SHARED((N_PAD, 16), _f32),
        pltpu.VMEM((8, 128), _i32),
        pltpu.VMEM((8, 128), _i32),
        pltpu.VMEM((1024, 16), _f32),
        pltpu.SemaphoreType.DMA,
        pltpu.SemaphoreType.DMA,
    ],
)
def _seg1_sc(src_hbm, dst_hbm, u16_hbm, out_hbm, acc, sblk, dblk, rows, sem,
             sem2):
    c = lax.axis_index("c")
    s = lax.axis_index("s")
    zeros = jnp.zeros((16,), _f32)

    def _zb(i, _):
        rows[i, :] = zeros
        return _
    lax.fori_loop(0, 782, _zb, None)

    zr = s * (N_PAD // 16)

    def _za(k, _):
        pltpu.sync_copy(
            rows.at[pl.ds(0, 782), :], acc.at[pl.ds(zr + k * 782, 782), :])
        return _
    lax.fori_loop(0, 8, _za, None)
    plsc.subcore_barrier()

    rows_per_tile = EROWS // 32
    base = c * (EROWS // 2) + s * rows_per_tile

    def _block(b, _):
        r0 = base + b * 8
        pltpu.sync_copy(src_hbm.at[pl.ds(r0, 8), :], sblk)
        pltpu.sync_copy(dst_hbm.at[pl.ds(r0, 8), :], dblk)
        hs = [
            pltpu.async_copy(
                u16_hbm.at[sblk.at[g]], rows.at[pl.ds(g * 128, 128), :], sem)
            for g in range(8)
        ]
        for h in hs:
            h.wait()
        ws = [
            pltpu.async_copy(
                rows.at[pl.ds(g * 128, 128), :], acc.at[dblk.at[g]], sem2,
                add=True)
            for g in range(8)
        ]
        for w in ws:
            w.wait()
        return _
    lax.fori_loop(0, rows_per_tile // 8, _block, None)
    plsc.subcore_barrier()

    pltpu.sync_copy(
        acc.at[pl.ds(zr, N_PAD // 16), :],
        out_hbm.at[c, pl.ds(zr, N_PAD // 16), :])


@functools.partial(
    pl.kernel,
    out_type=jax.ShapeDtypeStruct((4 * CHUNK, 64), _f32),
    mesh=_mesh,
    compiler_params=pltpu.CompilerParams(
        needs_layout_passes=False, use_tc_tiling_on_sc=False),
    scratch_types=[
        pltpu.VMEM_SHARED((ACC2, 64), _f32),
        pltpu.VMEM((8, 128), _i32),
        pltpu.VMEM((8, 128), _i32),
        pltpu.VMEM((CAP,), _i32),
        pltpu.VMEM((CAP,), _i32),
        pltpu.VMEM((2, 128), _i32),
        pltpu.VMEM((256, 64), _f32),
        pltpu.SemaphoreType.DMA,
        pltpu.SemaphoreType.DMA,
    ],
)
def _seg2_sc(src_hbm, dst_hbm, u2_hbm, out_hbm,
             acc, sblk, dblk, csrc, cdst, didx, rowbuf, sem, sc0):
    c = lax.axis_index("c")
    s = lax.axis_index("s")
    zeros = jnp.zeros((16,), _f32)

    rows_per_tile = EROWS // 16
    base = s * rows_per_tile

    for p in range(2):
        lo = (c + 2 * p) * CHUNK

        def _zb(i, _):
            g = i // 4
            k = i % 4
            rowbuf[g, pl.ds(k * 16, 16)] = zeros
            return _
        lax.fori_loop(0, 64 * 4, _zb, None)

        def _za(k, _):
            pltpu.sync_copy(
                rowbuf.at[pl.ds(0, 64), :],
                acc.at[pl.ds(s * (CHUNK // 16) + k * 64, 64), :])
            return _
        lax.fori_loop(0, 25, _za, None)
        plsc.subcore_barrier()

        def _block(b, cnt):
            r0 = base + b * 8
            pltpu.sync_copy(src_hbm.at[pl.ds(r0, 8), :], sblk)
            pltpu.sync_copy(dst_hbm.at[pl.ds(r0, 8), :], dblk)

            def _comp(j, cnt):
                g = j // 8
                k = j % 8
                d = dblk[g, pl.ds(k * 16, 16)]
                sv = sblk[g, pl.ds(k * 16, 16)]
                inm = (d >= lo) & (d < lo + CHUNK)
                plsc.store_compressed(csrc.at[pl.ds(cnt, 16)], sv, mask=inm)
                plsc.store_compressed(cdst.at[pl.ds(cnt, 16)], d - lo, mask=inm)
                pcv = plsc.all_reduce_population_count(inm)
                pc = pcv if pcv.ndim == 0 else pcv[0]
                return cnt + pc
            cnt = lax.fori_loop(0, 64, _comp, cnt)

            nfull = lax.shift_right_logical(cnt, 7)

            def _pair(rp, _):
                offa = pl.multiple_of(rp * 256, 128)
                offb = offa + 128
                ha = pltpu.async_copy(
                    u2_hbm.at[csrc.at[pl.ds(offa, 128)]],
                    rowbuf.at[pl.ds(0, 128), :], sem)
                hb = pltpu.async_copy(
                    u2_hbm.at[csrc.at[pl.ds(offb, 128)]],
                    rowbuf.at[pl.ds(128, 128), :], sem)

                def _fd(j, _):
                    didx[0, pl.ds(j * 16, 16)] = cdst[pl.ds(offa + j * 16, 16)]
                    didx[1, pl.ds(j * 16, 16)] = cdst[pl.ds(offb + j * 16, 16)]
                    return _
                lax.fori_loop(0, 8, _fd, None)
                ha.wait()
                hb.wait()
                wa = pltpu.async_copy(
                    rowbuf.at[pl.ds(0, 128), :], acc.at[didx.at[0]], sc0,
                    add=True)
                wb = pltpu.async_copy(
                    rowbuf.at[pl.ds(128, 128), :], acc.at[didx.at[1]], sc0,
                    add=True)
                wa.wait()
                wb.wait()
                return _
            lax.fori_loop(0, lax.shift_right_logical(nfull, 1), _pair, None)

            @pl.when((nfull & 1) == 1)
            def _tail():
                offt = pl.multiple_of((nfull - 1) * 128, 128)
                ht = pltpu.async_copy(
                    u2_hbm.at[csrc.at[pl.ds(offt, 128)]],
                    rowbuf.at[pl.ds(0, 128), :], sem)

                def _fd(j, _):
                    didx[0, pl.ds(j * 16, 16)] = cdst[pl.ds(offt + j * 16, 16)]
                    return _
                lax.fori_loop(0, 8, _fd, None)
                ht.wait()
                pltpu.sync_copy(
                    rowbuf.at[pl.ds(0, 128), :], acc.at[didx.at[0]], add=True)

            @pl.when(nfull > 0)
            def _mv():
                def _mvv(j, _):
                    csrc[pl.ds(j * 16, 16)] = csrc[pl.ds(nfull * 128 + j * 16, 16)]
                    cdst[pl.ds(j * 16, 16)] = cdst[pl.ds(nfull * 128 + j * 16, 16)]
                    return _
                lax.fori_loop(0, 8, _mvv, None)
            return cnt & 127
        cnt = lax.fori_loop(0, rows_per_tile // 8, _block, jnp.int32(0))

        @pl.when(cnt > 0)
        def _flush():
            def _pad(j, _):
                csrc[pl.ds(cnt + j * 16, 16)] = jnp.zeros((16,), _i32)
                cdst[pl.ds(cnt + j * 16, 16)] = jnp.full((16,), CHUNK, _i32)
                return _
            lax.fori_loop(0, 8, _pad, None)
            hf = pltpu.async_copy(
                u2_hbm.at[csrc.at[pl.ds(0, 128)]],
                rowbuf.at[pl.ds(0, 128), :], sem)

            def _fd(j, _):
                didx[0, pl.ds(j * 16, 16)] = cdst[pl.ds(j * 16, 16)]
                return _
            lax.fori_loop(0, 8, _fd, None)
            hf.wait()
            pltpu.sync_copy(
                rowbuf.at[pl.ds(0, 128), :], acc.at[didx.at[0]], add=True)

        plsc.subcore_barrier()
        pltpu.sync_copy(
            acc.at[pl.ds(s * (CHUNK // 16), CHUNK // 16), :],
            out_hbm.at[pl.ds(lo + s * (CHUNK // 16), CHUNK // 16), :])
        plsc.subcore_barrier()


def _stage_a0(dp_ref, deg_ref):
    deg_ref[...] = jnp.sum(dp_ref[...], axis=0)[:, None]


def _stage_a(deg_ref, x_ref, dis_ref, u16_ref):
    deg = deg_ref[...][:, 0] + 1.0
    dis = lax.rsqrt(deg)
    dis_ref[...] = dis[:, None]
    u16_ref[...] = jnp.concatenate(
        [dis[:, None] * x_ref[...], jnp.zeros((R, 14), _f32)], axis=1)


def _stage_b(t1_ref, u16_ref, dis_ref, w1_ref, b1_ref, w2_ref, u2_ref):
    t1 = t1_ref[0] + t1_ref[1]
    dis = dis_ref[...]
    s1 = dis * (t1[:, :2] + u16_ref[..., :2])
    h1 = jnp.maximum(
        jnp.dot(s1, w1_ref[...], preferred_element_type=_f32) + b1_ref[...],
        0.0)
    z = jnp.dot(h1, w2_ref[...], preferred_element_type=_f32)
    u2_ref[...] = dis * z


def _stage_c(t2_ref, u2_ref, dis_ref, b2_ref, wl_ref, bl_ref, out_ref, acc):
    i = pl.program_id(0)
    h2 = jnp.maximum(
        dis_ref[...] * (t2_ref[...] + u2_ref[...]) + b2_ref[...], 0.0)
    part = jnp.sum(h2, axis=0, keepdims=True)

    @pl.when(i == 0)
    def _():
        acc[...] = part

    @pl.when(i > 0)
    def _():
        acc[...] = acc[...] + part

    @pl.when(i == GRID - 1)
    def _():
        pooled = acc[...] / float(N_NODES)
        out_ref[...] = (
            jnp.dot(pooled, wl_ref[...], preferred_element_type=_f32)
            + bl_ref[...])


def kernel(x, edge_index, W1, b1, W2, b2, Wl, bl):
    ei = edge_index.astype(_i32)
    src = jnp.concatenate([ei[0], jnp.zeros((PAD_E,), _i32)])
    dst = jnp.concatenate([ei[1], jnp.full((PAD_E,), N_NODES, _i32)])
    src2d = src.reshape(EROWS, 128)
    dst2d = dst.reshape(EROWS, 128)

    degpart = _deg_sc(dst2d)

    deg2d = pl.pallas_call(
        _stage_a0,
        grid=(17,),
        in_specs=[pl.BlockSpec((32, 5888), lambda i: (0, i))],
        out_specs=pl.BlockSpec((5888, 1), lambda i: (i, 0)),
        out_shape=jax.ShapeDtypeStruct((N_PAD, 1), _f32),
    )(degpart)

    dis, u16 = pl.pallas_call(
        _stage_a,
        grid=(GRID,),
        in_specs=[
            pl.BlockSpec((R, 1), lambda i: (i, 0)),
            pl.BlockSpec((R, 2), lambda i: (i, 0)),
        ],
        out_specs=[
            pl.BlockSpec((R, 1), lambda i: (i, 0)),
            pl.BlockSpec((R, 16), lambda i: (i, 0)),
        ],
        out_shape=[
            jax.ShapeDtypeStruct((N_NODES, 1), _f32),
            jax.ShapeDtypeStruct((N_NODES, 16), _f32),
        ],
    )(deg2d, x)

    t1part = _seg1_sc(src2d, dst2d, u16)

    u2 = pl.pallas_call(
        _stage_b,
        grid=(GRID,),
        in_specs=[
            pl.BlockSpec((2, R, 16), lambda i: (0, i, 0)),
            pl.BlockSpec((R, 16), lambda i: (i, 0)),
            pl.BlockSpec((R, 1), lambda i: (i, 0)),
            pl.BlockSpec((2, 64), lambda i: (0, 0)),
            pl.BlockSpec((1, 64), lambda i: (0, 0)),
            pl.BlockSpec((64, 64), lambda i: (0, 0)),
        ],
        out_specs=pl.BlockSpec((R, 64), lambda i: (i, 0)),
        out_shape=jax.ShapeDtypeStruct((N_NODES, 64), _f32),
    )(t1part, u16, dis, W1, b1.reshape(1, 64), W2)

    t2 = _seg2_sc(src2d, dst2d, u2)

    out = pl.pallas_call(
        _stage_c,
        grid=(GRID,),
        in_specs=[
            pl.BlockSpec((R, 64), lambda i: (i, 0)),
            pl.BlockSpec((R, 64), lambda i: (i, 0)),
            pl.BlockSpec((R, 1), lambda i: (i, 0)),
            pl.BlockSpec((1, 64), lambda i: (0, 0)),
            pl.BlockSpec((64, 1), lambda i: (0, 0)),
            pl.BlockSpec((1, 1), lambda i: (0, 0)),
        ],
        out_specs=pl.BlockSpec((1, 1), lambda i: (0, 0)),
        out_shape=jax.ShapeDtypeStruct((1, 1), _f32),
        scratch_shapes=[pltpu.VMEM((1, 64), _f32)],
    )(t2, u2, dis, b2.reshape(1, 64), Wl, bl.reshape(1, 1))

    return out.reshape(1)


# speedup vs baseline: 1.0287x; 1.0048x over previous
import functools

import jax
import jax.numpy as jnp
from jax import lax
from jax.experimental import pallas as pl
from jax.experimental.pallas import tpu as pltpu
from jax.experimental.pallas import tpu_sc as plsc

N_NODES = 100000
N_EDGES = 6400000
N_PAD = 100096
E_PAD = 6422528
PAD_E = E_PAD - N_EDGES
EROWS = E_PAD // 128
CHUNK = 25600
ACC2 = CHUNK + 8
CAP = 1280
CAP2 = 2064
SPG = 400
R = 2000
GRID = N_NODES // R

_mesh = plsc.VectorSubcoreMesh(core_axis_name="c", subcore_axis_name="s")
_f32 = jnp.float32
_i32 = jnp.int32


@functools.partial(
    pl.kernel,
    out_type=jax.ShapeDtypeStruct((32, N_PAD), _f32),
    mesh=_mesh,
    compiler_params=pltpu.CompilerParams(needs_layout_passes=False),
    scratch_types=[
        pltpu.VMEM((N_PAD,), _f32),
        pltpu.VMEM((32, 128), _i32),
    ],
)
def _deg_sc(dst_hbm, out_hbm, acc, blk):
    c = lax.axis_index("c")
    s = lax.axis_index("s")
    wid = c * 16 + s
    zeros = jnp.zeros((16,), _f32)
    ones = jnp.ones((16,), _f32)

    def _zero(i, _):
        acc[pl.ds(i * 16, 16)] = zeros
        return _
    lax.fori_loop(0, N_PAD // 16, _zero, None)

    rows_per_tile = EROWS // 32
    base = wid * rows_per_tile

    def _block(b, _):
        pltpu.sync_copy(dst_hbm.at[pl.ds(base + b * 32, 32), :], blk)

        def _hist(j, _):
            g = j // 8
            k = j % 8
            idx = blk[g, pl.ds(k * 16, 16)]
            plsc.addupdate_scatter(acc, [idx], ones)
            return _
        lax.fori_loop(0, 32 * 8, _hist, None)
        return _
    lax.fori_loop(0, rows_per_tile // 32, _block, None)

    pltpu.sync_copy(acc, out_hbm.at[wid])


@functools.partial(
    pl.kernel,
    out_type=jax.ShapeDtypeStruct((2, N_PAD, 16), _f32),
    mesh=_mesh,
    compiler_params=pltpu.CompilerParams(
        needs_layout_passes=False, use_tc_tiling_on_sc=False),
    scratch_types=[
        pltpu.VMEM_SHARED((N_PAD, 16), _f32),
        pltpu.VMEM((8, 128), _i32),
        pltpu.VMEM((8, 128), _i32),
        pltpu.VMEM((1024, 16), _f32),
        pltpu.SemaphoreType.DMA,
        pltpu.SemaphoreType.DMA,
    ],
)
def _seg1_sc(src_hbm, dst_hbm, u16_hbm, out_hbm, acc, sblk, dblk, rows, sem,
             sem2):
    c = lax.axis_index("c")
    s = lax.axis_index("s")
    zeros = jnp.zeros((16,), _f32)

    def _zb(i, _):
        rows[i, :] = zeros
        return _
    lax.fori_loop(0, 782, _zb, None)

    zr = s * (N_PAD // 16)

    def _za(k, _):
        pltpu.sync_copy(
            rows.at[pl.ds(0, 782), :], acc.at[pl.ds(zr + k * 782, 782), :])
        return _
    lax.fori_loop(0, 8, _za, None)
    plsc.subcore_barrier()

    rows_per_tile = EROWS // 32
    base = c * (EROWS // 2) + s * rows_per_tile

    def _block(b, _):
        r0 = base + b * 8
        pltpu.sync_copy(src_hbm.at[pl.ds(r0, 8), :], sblk)
        pltpu.sync_copy(dst_hbm.at[pl.ds(r0, 8), :], dblk)
        hs = [
            pltpu.async_copy(
                u16_hbm.at[sblk.at[g]], rows.at[pl.ds(g * 128, 128), :], sem)
            for g in range(8)
        ]
        for h in hs:
            h.wait()
        ws = [
            pltpu.async_copy(
                rows.at[pl.ds(g * 128, 128), :], acc.at[dblk.at[g]], sem2,
                add=True)
            for g in range(8)
        ]
        for w in ws:
            w.wait()
        return _
    lax.fori_loop(0, rows_per_tile // 8, _block, None)
    plsc.subcore_barrier()

    pltpu.sync_copy(
        acc.at[pl.ds(zr, N_PAD // 16), :],
        out_hbm.at[c, pl.ds(zr, N_PAD // 16), :])


@functools.partial(
    pl.kernel,
    out_type=[
        jax.ShapeDtypeStruct((4 * CHUNK, 64), _f32),
        jax.ShapeDtypeStruct((2, 16, SPG * 1024), _i32),
        jax.ShapeDtypeStruct((2, 16, SPG * 1024), _i32),
    ],
    mesh=_mesh,
    compiler_params=pltpu.CompilerParams(
        needs_layout_passes=False, use_tc_tiling_on_sc=False),
    scratch_types=[
        pltpu.VMEM_SHARED((ACC2, 64), _f32),
        pltpu.VMEM((8, 128), _i32),
        pltpu.VMEM((8, 128), _i32),
        pltpu.VMEM((CAP,), _i32),
        pltpu.VMEM((CAP,), _i32),
        pltpu.VMEM((CAP2,), _i32),
        pltpu.VMEM((CAP2,), _i32),
        pltpu.VMEM((2, 128), _i32),
        pltpu.VMEM((256, 64), _f32),
        pltpu.SemaphoreType.DMA,
    ],
)
def _seg2_sc(src_hbm, dst_hbm, u2_hbm, out_hbm, spsrc_hbm, spdl_hbm,
             acc, sblk, dblk, csrc, cdst, csp, cdp, didx, rowbuf, sem):
    c = lax.axis_index("c")
    s = lax.axis_index("s")
    zeros = jnp.zeros((16,), _f32)
    izeros = jnp.zeros((16,), _i32)
    idump = jnp.full((16,), CHUNK, _i32)

    rows_per_tile = EROWS // 16
    base = s * rows_per_tile
    lo = c * CHUNK
    lo2 = (c + 2) * CHUNK

    def _zero_acc():
        def _zb(i, _):
            g = i // 4
            k = i % 4
            rowbuf[g, pl.ds(k * 16, 16)] = zeros
            return _
        lax.fori_loop(0, 64 * 4, _zb, None)

        def _za(k, _):
            pltpu.sync_copy(
                rowbuf.at[pl.ds(0, 64), :],
                acc.at[pl.ds(s * (CHUNK // 16) + k * 64, 64), :])
            return _
        lax.fori_loop(0, 25, _za, None)

    def _fill_didx(dlbuf, off, slot):
        def _fd(j, _):
            didx[slot, pl.ds(j * 16, 16)] = dlbuf[pl.ds(off + j * 16, 16)]
            return _
        lax.fori_loop(0, 8, _fd, None)

    def _drain_pairs(nfull, idxbuf, dlbuf):
        def _pair(rp, _):
            offa = pl.multiple_of(rp * 256, 128)
            offb = offa + 128
            ha = pltpu.async_copy(
                u2_hbm.at[idxbuf.at[pl.ds(offa, 128)]],
                rowbuf.at[pl.ds(0, 128), :], sem)
            hb = pltpu.async_copy(
                u2_hbm.at[idxbuf.at[pl.ds(offb, 128)]],
                rowbuf.at[pl.ds(128, 128), :], sem)
            _fill_didx(dlbuf, offa, 0)
            _fill_didx(dlbuf, offb, 1)
            ha.wait()
            hb.wait()
            pltpu.sync_copy(
                rowbuf.at[pl.ds(0, 128), :], acc.at[didx.at[0]], add=True)
            pltpu.sync_copy(
                rowbuf.at[pl.ds(128, 128), :], acc.at[didx.at[1]], add=True)
            return _
        lax.fori_loop(0, lax.shift_right_logical(nfull, 1), _pair, None)

        @pl.when((nfull & 1) == 1)
        def _tail():
            offt = pl.multiple_of((nfull - 1) * 128, 128)
            ht = pltpu.async_copy(
                u2_hbm.at[idxbuf.at[pl.ds(offt, 128)]],
                rowbuf.at[pl.ds(0, 128), :], sem)
            _fill_didx(dlbuf, offt, 0)
            ht.wait()
            pltpu.sync_copy(
                rowbuf.at[pl.ds(0, 128), :], acc.at[didx.at[0]], add=True)

    _zero_acc()
    plsc.subcore_barrier()

    def _block(b, st):
        cnt, cnt2, spr = st
        r0 = base + b * 8
        pltpu.sync_copy(src_hbm.at[pl.ds(r0, 8), :], sblk)
        pltpu.sync_copy(dst_hbm.at[pl.ds(r0, 8), :], dblk)

        def _comp(j, st2):
            cnt, cnt2 = st2
            g = j // 8
            k = j % 8
            d = dblk[g, pl.ds(k * 16, 16)]
            sv = sblk[g, pl.ds(k * 16, 16)]
            inm = (d >= lo) & (d < lo + CHUNK)
            inm2 = (d >= lo2) & (d < lo2 + CHUNK)
            plsc.store_compressed(csrc.at[pl.ds(cnt, 16)], sv, mask=inm)
            plsc.store_compressed(cdst.at[pl.ds(cnt, 16)], d - lo, mask=inm)
            plsc.store_compressed(csp.at[pl.ds(cnt2, 16)], sv, mask=inm2)
            plsc.store_compressed(cdp.at[pl.ds(cnt2, 16)], d - lo2, mask=inm2)
            p1 = plsc.all_reduce_population_count(inm)
            p1 = p1 if p1.ndim == 0 else p1[0]
            p2 = plsc.all_reduce_population_count(inm2)
            p2 = p2 if p2.ndim == 0 else p2[0]
            return (cnt + p1, cnt2 + p2)
        cnt, cnt2 = lax.fori_loop(0, 64, _comp, (cnt, cnt2))

        nfull = lax.shift_right_logical(cnt, 7)
        _drain_pairs(nfull, csrc, cdst)

        @pl.when(nfull > 0)
        def _mv():
            def _mvv(j, _):
                csrc[pl.ds(j * 16, 16)] = csrc[pl.ds(nfull * 128 + j * 16, 16)]
                cdst[pl.ds(j * 16, 16)] = cdst[pl.ds(nfull * 128 + j * 16, 16)]
                return _
            lax.fori_loop(0, 8, _mvv, None)

        ng = lax.shift_right_logical(cnt2, 10)

        def _spill(f, spr):
            off = pl.multiple_of(f * 1024, 1024)
            pltpu.sync_copy(
                csp.at[pl.ds(off, 1024)],
                spsrc_hbm.at[c, s, pl.ds(spr * 1024, 1024)])
            pltpu.sync_copy(
                cdp.at[pl.ds(off, 1024)],
                spdl_hbm.at[c, s, pl.ds(spr * 1024, 1024)])
            return spr + 1
        spr = lax.fori_loop(0, ng, _spill, spr)

        @pl.when(ng > 0)
        def _mv2():
            def _mvv(j, _):
                csp[pl.ds(j * 16, 16)] = csp[pl.ds(ng * 1024 + j * 16, 16)]
                cdp[pl.ds(j * 16, 16)] = cdp[pl.ds(ng * 1024 + j * 16, 16)]
                return _
            lax.fori_loop(0, 64, _mvv, None)
        return (cnt & 127, cnt2 & 1023, spr)

    cnt, cnt2, spr = lax.fori_loop(
        0, rows_per_tile // 8, _block,
        (jnp.int32(0), jnp.int32(0), jnp.int32(0)))

    @pl.when(cnt > 0)
    def _flush():
        def _pad(j, _):
            csrc[pl.ds(cnt + j * 16, 16)] = izeros
            cdst[pl.ds(cnt + j * 16, 16)] = idump
            return _
        lax.fori_loop(0, 8, _pad, None)
        hf = pltpu.async_copy(
            u2_hbm.at[csrc.at[pl.ds(0, 128)]],
            rowbuf.at[pl.ds(0, 128), :], sem)
        _fill_didx(cdst, 0, 0)
        hf.wait()
        pltpu.sync_copy(
            rowbuf.at[pl.ds(0, 128), :], acc.at[didx.at[0]], add=True)

    @pl.when(cnt2 > 0)
    def _flush2():
        def _pad(j, _):
            csp[pl.ds(cnt2 + j * 16, 16)] = izeros
            cdp[pl.ds(cnt2 + j * 16, 16)] = idump
            return _
        lax.fori_loop(0, 64, _pad, None)
        pltpu.sync_copy(
            csp.at[pl.ds(0, 1024)],
            spsrc_hbm.at[c, s, pl.ds(spr * 1024, 1024)])
        pltpu.sync_copy(
            cdp.at[pl.ds(0, 1024)],
            spdl_hbm.at[c, s, pl.ds(spr * 1024, 1024)])
    spr = spr + jnp.where(cnt2 > 0, 1, 0).astype(jnp.int32)

    plsc.subcore_barrier()
    pltpu.sync_copy(
        acc.at[pl.ds(s * (CHUNK // 16), CHUNK // 16), :],
        out_hbm.at[pl.ds(lo + s * (CHUNK // 16), CHUNK // 16), :])
    plsc.subcore_barrier()

    _zero_acc()
    plsc.subcore_barrier()

    def _grp(g, _):
        pltpu.sync_copy(
            spsrc_hbm.at[c, s, pl.ds(g * 1024, 1024)], csp.at[pl.ds(0, 1024)])
        pltpu.sync_copy(
            spdl_hbm.at[c, s, pl.ds(g * 1024, 1024)], cdp.at[pl.ds(0, 1024)])
        _drain_pairs(jnp.int32(8), csp, cdp)
        return _
    lax.fori_loop(0, spr, _grp, None)

    plsc.subcore_barrier()
    pltpu.sync_copy(
        acc.at[pl.ds(s * (CHUNK // 16), CHUNK // 16), :],
        out_hbm.at[pl.ds(lo2 + s * (CHUNK // 16), CHUNK // 16), :])
    plsc.subcore_barrier()


def _stage_a0(dp_ref, deg_ref):
    deg_ref[...] = jnp.sum(dp_ref[...], axis=0)[:, None]


def _stage_a(deg_ref, x_ref, dis_ref, u16_ref):
    deg = deg_ref[...][:, 0] + 1.0
    dis = lax.rsqrt(deg)
    dis_ref[...] = dis[:, None]
    u16_ref[...] = jnp.concatenate(
        [dis[:, None] * x_ref[...], jnp.zeros((R, 14), _f32)], axis=1)


def _stage_b(t1_ref, u16_ref, dis_ref, w1_ref, b1_ref, w2_ref, u2_ref):
    t1 = t1_ref[0] + t1_ref[1]
    dis = dis_ref[...]
    s1 = dis * (t1[:, :2] + u16_ref[..., :2])
    h1 = jnp.maximum(
        jnp.dot(s1, w1_ref[...], preferred_element_type=_f32) + b1_ref[...],
        0.0)
    z = jnp.dot(h1, w2_ref[...], preferred_element_type=_f32)
    u2_ref[...] = dis * z


def _stage_c(t2_ref, u2_ref, dis_ref, b2_ref, wl_ref, bl_ref, out_ref, acc):
    i = pl.program_id(0)
    h2 = jnp.maximum(
        dis_ref[...] * (t2_ref[...] + u2_ref[...]) + b2_ref[...], 0.0)
    part = jnp.sum(h2, axis=0, keepdims=True)

    @pl.when(i == 0)
    def _():
        acc[...] = part

    @pl.when(i > 0)
    def _():
        acc[...] = acc[...] + part

    @pl.when(i == GRID - 1)
    def _():
        pooled = acc[...] / float(N_NODES)
        out_ref[...] = (
            jnp.dot(pooled, wl_ref[...], preferred_element_type=_f32)
            + bl_ref[...])


def kernel(x, edge_index, W1, b1, W2, b2, Wl, bl):
    ei = edge_index.astype(_i32)
    src = jnp.concatenate([ei[0], jnp.zeros((PAD_E,), _i32)])
    dst = jnp.concatenate([ei[1], jnp.full((PAD_E,), N_NODES, _i32)])
    src2d = src.reshape(EROWS, 128)
    dst2d = dst.reshape(EROWS, 128)

    degpart = _deg_sc(dst2d)

    deg2d = pl.pallas_call(
        _stage_a0,
        grid=(17,),
        in_specs=[pl.BlockSpec((32, 5888), lambda i: (0, i))],
        out_specs=pl.BlockSpec((5888, 1), lambda i: (i, 0)),
        out_shape=jax.ShapeDtypeStruct((N_PAD, 1), _f32),
    )(degpart)

    dis, u16 = pl.pallas_call(
        _stage_a,
        grid=(GRID,),
        in_specs=[
            pl.BlockSpec((R, 1), lambda i: (i, 0)),
            pl.BlockSpec((R, 2), lambda i: (i, 0)),
        ],
        out_specs=[
            pl.BlockSpec((R, 1), lambda i: (i, 0)),
            pl.BlockSpec((R, 16), lambda i: (i, 0)),
        ],
        out_shape=[
            jax.ShapeDtypeStruct((N_NODES, 1), _f32),
            jax.ShapeDtypeStruct((N_NODES, 16), _f32),
        ],
    )(deg2d, x)

    t1part = _seg1_sc(src2d, dst2d, u16)

    u2 = pl.pallas_call(
        _stage_b,
        grid=(GRID,),
        in_specs=[
            pl.BlockSpec((2, R, 16), lambda i: (0, i, 0)),
            pl.BlockSpec((R, 16), lambda i: (i, 0)),
            pl.BlockSpec((R, 1), lambda i: (i, 0)),
            pl.BlockSpec((2, 64), lambda i: (0, 0)),
            pl.BlockSpec((1, 64), lambda i: (0, 0)),
            pl.BlockSpec((64, 64), lambda i: (0, 0)),
        ],
        out_specs=pl.BlockSpec((R, 64), lambda i: (i, 0)),
        out_shape=jax.ShapeDtypeStruct((N_NODES, 64), _f32),
    )(t1part, u16, dis, W1, b1.reshape(1, 64), W2)

    t2, _sp1, _sp2 = _seg2_sc(src2d, dst2d, u2)

    out = pl.pallas_call(
        _stage_c,
        grid=(GRID,),
        in_specs=[
            pl.BlockSpec((R, 64), lambda i: (i, 0)),
            pl.BlockSpec((R, 64), lambda i: (i, 0)),
            pl.BlockSpec((R, 1), lambda i: (i, 0)),
            pl.BlockSpec((1, 64), lambda i: (0, 0)),
            pl.BlockSpec((64, 1), lambda i: (0, 0)),
            pl.BlockSpec((1, 1), lambda i: (0, 0)),
        ],
        out_specs=pl.BlockSpec((1, 1), lambda i: (0, 0)),
        out_shape=jax.ShapeDtypeStruct((1, 1), _f32),
        scratch_shapes=[pltpu.VMEM((1, 64), _f32)],
    )(t2, u2, dis, b2.reshape(1, 64), Wl, bl.reshape(1, 1))

    return out.reshape(1)


# speedup vs baseline: 1.2020x; 1.1685x over previous
import functools

import jax
import jax.numpy as jnp
from jax import lax
from jax.experimental import pallas as pl
from jax.experimental.pallas import tpu as pltpu
from jax.experimental.pallas import tpu_sc as plsc

N_NODES = 100000
N_EDGES = 6400000
N_PAD = 100096
E_PAD = 6422528
PAD_E = E_PAD - N_EDGES
EROWS = E_PAD // 128
CHUNK = 25600
ACC2 = CHUNK + 8
CAP = 1280
CAP2 = 2064
SPG = 400
R = 2000
GRID = N_NODES // R

_mesh = plsc.VectorSubcoreMesh(core_axis_name="c", subcore_axis_name="s")
_f32 = jnp.float32
_i32 = jnp.int32


@functools.partial(
    pl.kernel,
    out_type=jax.ShapeDtypeStruct((32, N_PAD), _f32),
    mesh=_mesh,
    compiler_params=pltpu.CompilerParams(needs_layout_passes=False),
    scratch_types=[
        pltpu.VMEM((N_PAD,), _f32),
        pltpu.VMEM((32, 128), _i32),
    ],
)
def _deg_sc(dst_hbm, out_hbm, acc, blk):
    c = lax.axis_index("c")
    s = lax.axis_index("s")
    wid = c * 16 + s
    zeros = jnp.zeros((16,), _f32)
    ones = jnp.ones((16,), _f32)

    def _zero(i, _):
        acc[pl.ds(i * 16, 16)] = zeros
        return _
    lax.fori_loop(0, N_PAD // 16, _zero, None)

    rows_per_tile = EROWS // 32
    base = wid * rows_per_tile

    def _block(b, _):
        pltpu.sync_copy(dst_hbm.at[pl.ds(base + b * 32, 32), :], blk)

        def _hist(j, _):
            g = j // 8
            k = j % 8
            idx = blk[g, pl.ds(k * 16, 16)]
            plsc.addupdate_scatter(acc, [idx], ones)
            return _
        lax.fori_loop(0, 32 * 8, _hist, None)
        return _
    lax.fori_loop(0, rows_per_tile // 32, _block, None)

    pltpu.sync_copy(acc, out_hbm.at[wid])


@functools.partial(
    pl.kernel,
    out_type=jax.ShapeDtypeStruct((2, N_PAD, 16), _f32),
    mesh=_mesh,
    compiler_params=pltpu.CompilerParams(
        needs_layout_passes=False, use_tc_tiling_on_sc=False),
    scratch_types=[
        pltpu.VMEM_SHARED((N_PAD, 16), _f32),
        pltpu.VMEM((8, 128), _i32),
        pltpu.VMEM((8, 128), _i32),
        pltpu.VMEM((8, 128), _i32),
        pltpu.VMEM((8, 128), _i32),
        pltpu.VMEM((1024, 16), _f32),
        pltpu.SemaphoreType.DMA,
        pltpu.SemaphoreType.DMA,
        pltpu.SemaphoreType.DMA,
    ],
)
def _seg1_sc(src_hbm, dst_hbm, u16_hbm, out_hbm, acc, sblk, dblk, sblk2,
             dblk2, rows, sem, sem2, seme):
    c = lax.axis_index("c")
    s = lax.axis_index("s")
    zeros = jnp.zeros((16,), _f32)

    def _zb(i, _):
        rows[i, :] = zeros
        return _
    lax.fori_loop(0, 782, _zb, None)

    zr = s * (N_PAD // 16)

    def _za(k, _):
        pltpu.sync_copy(
            rows.at[pl.ds(0, 782), :], acc.at[pl.ds(zr + k * 782, 782), :])
        return _
    lax.fori_loop(0, 8, _za, None)
    plsc.subcore_barrier()

    rows_per_tile = EROWS // 32
    nblk = rows_per_tile // 8
    base = c * (EROWS // 2) + s * rows_per_tile

    def _stage(b, sb, db):
        r0 = base + b * 8
        pltpu.async_copy(src_hbm.at[pl.ds(r0, 8), :], sb, seme)
        pltpu.async_copy(dst_hbm.at[pl.ds(r0, 8), :], db, seme)

    def _do_block(b, sb, db, sbn, dbn):
        r0 = base + b * 8
        pltpu.make_async_copy(src_hbm.at[pl.ds(r0, 8), :], sb, seme).wait()
        pltpu.make_async_copy(dst_hbm.at[pl.ds(r0, 8), :], db, seme).wait()

        @pl.when(b + 1 < nblk)
        def _():
            _stage(b + 1, sbn, dbn)
        hs = [
            pltpu.async_copy(
                u16_hbm.at[sb.at[g]], rows.at[pl.ds(g * 128, 128), :], sem)
            for g in range(8)
        ]
        for h in hs:
            h.wait()
        ws = [
            pltpu.async_copy(
                rows.at[pl.ds(g * 128, 128), :], acc.at[db.at[g]], sem2,
                add=True)
            for g in range(8)
        ]
        for w in ws:
            w.wait()

    _stage(0, sblk, dblk)

    def _block2(b2, _):
        _do_block(2 * b2, sblk, dblk, sblk2, dblk2)
        _do_block(2 * b2 + 1, sblk2, dblk2, sblk, dblk)
        return _
    lax.fori_loop(0, nblk // 2, _block2, None)
    plsc.subcore_barrier()

    pltpu.sync_copy(
        acc.at[pl.ds(zr, N_PAD // 16), :],
        out_hbm.at[c, pl.ds(zr, N_PAD // 16), :])


@functools.partial(
    pl.kernel,
    out_type=[
        jax.ShapeDtypeStruct((4 * CHUNK, 64), _f32),
        jax.ShapeDtypeStruct((2, 16, SPG * 1024), _i32),
        jax.ShapeDtypeStruct((2, 16, SPG * 1024), _i32),
    ],
    mesh=_mesh,
    compiler_params=pltpu.CompilerParams(
        needs_layout_passes=False, use_tc_tiling_on_sc=False),
    scratch_types=[
        pltpu.VMEM_SHARED((ACC2, 64), _f32),
        pltpu.VMEM((8, 128), _i32),
        pltpu.VMEM((8, 128), _i32),
        pltpu.VMEM((8, 128), _i32),
        pltpu.VMEM((8, 128), _i32),
        pltpu.VMEM((CAP,), _i32),
        pltpu.VMEM((CAP,), _i32),
        pltpu.VMEM((CAP2,), _i32),
        pltpu.VMEM((CAP2,), _i32),
        pltpu.VMEM((2, 128), _i32),
        pltpu.VMEM((256, 64), _f32),
        pltpu.SemaphoreType.DMA,
        pltpu.SemaphoreType.DMA,
    ],
)
def _seg2_sc(src_hbm, dst_hbm, u2_hbm, out_hbm, spsrc_hbm, spdl_hbm,
             acc, sblk, dblk, sblk2, dblk2, csrc, cdst, csp, cdp, didx,
             rowbuf, sem, seme):
    c = lax.axis_index("c")
    s = lax.axis_index("s")
    zeros = jnp.zeros((16,), _f32)
    izeros = jnp.zeros((16,), _i32)
    idump = jnp.full((16,), CHUNK, _i32)

    rows_per_tile = EROWS // 16
    base = s * rows_per_tile
    lo = c * CHUNK
    lo2 = (c + 2) * CHUNK

    def _zero_acc():
        def _zb(i, _):
            g = i // 4
            k = i % 4
            rowbuf[g, pl.ds(k * 16, 16)] = zeros
            return _
        lax.fori_loop(0, 64 * 4, _zb, None)

        def _za(k, _):
            pltpu.sync_copy(
                rowbuf.at[pl.ds(0, 64), :],
                acc.at[pl.ds(s * (CHUNK // 16) + k * 64, 64), :])
            return _
        lax.fori_loop(0, 25, _za, None)

    def _fill_didx(dlbuf, off, slot):
        def _fd(j, _):
            didx[slot, pl.ds(j * 16, 16)] = dlbuf[pl.ds(off + j * 16, 16)]
            return _
        lax.fori_loop(0, 8, _fd, None)

    def _drain_pairs(nfull, idxbuf, dlbuf):
        def _pair(rp, _):
            offa = pl.multiple_of(rp * 256, 128)
            offb = offa + 128
            ha = pltpu.async_copy(
                u2_hbm.at[idxbuf.at[pl.ds(offa, 128)]],
                rowbuf.at[pl.ds(0, 128), :], sem)
            hb = pltpu.async_copy(
                u2_hbm.at[idxbuf.at[pl.ds(offb, 128)]],
                rowbuf.at[pl.ds(128, 128), :], sem)
            _fill_didx(dlbuf, offa, 0)
            _fill_didx(dlbuf, offb, 1)
            ha.wait()
            hb.wait()
            pltpu.sync_copy(
                rowbuf.at[pl.ds(0, 128), :], acc.at[didx.at[0]], add=True)
            pltpu.sync_copy(
                rowbuf.at[pl.ds(128, 128), :], acc.at[didx.at[1]], add=True)
            return _
        lax.fori_loop(0, lax.shift_right_logical(nfull, 1), _pair, None)

        @pl.when((nfull & 1) == 1)
        def _tail():
            offt = pl.multiple_of((nfull - 1) * 128, 128)
            ht = pltpu.async_copy(
                u2_hbm.at[idxbuf.at[pl.ds(offt, 128)]],
                rowbuf.at[pl.ds(0, 128), :], sem)
            _fill_didx(dlbuf, offt, 0)
            ht.wait()
            pltpu.sync_copy(
                rowbuf.at[pl.ds(0, 128), :], acc.at[didx.at[0]], add=True)

    _zero_acc()
    plsc.subcore_barrier()

    nblk = rows_per_tile // 8

    def _stage(b, sb, db):
        r0 = base + b * 8
        pltpu.async_copy(src_hbm.at[pl.ds(r0, 8), :], sb, seme)
        pltpu.async_copy(dst_hbm.at[pl.ds(r0, 8), :], db, seme)

    def _do_block(b, st, sb, db, sbn, dbn):
        cnt, cnt2, spr = st
        r0 = base + b * 8
        pltpu.make_async_copy(src_hbm.at[pl.ds(r0, 8), :], sb, seme).wait()
        pltpu.make_async_copy(dst_hbm.at[pl.ds(r0, 8), :], db, seme).wait()

        @pl.when(b + 1 < nblk)
        def _():
            _stage(b + 1, sbn, dbn)

        def _comp(j, st2):
            cnt, cnt2 = st2
            g = j // 8
            k = j % 8
            d = db[g, pl.ds(k * 16, 16)]
            sv = sb[g, pl.ds(k * 16, 16)]
            inm = (d >= lo) & (d < lo + CHUNK)
            inm2 = (d >= lo2) & (d < lo2 + CHUNK)
            plsc.store_compressed(csrc.at[pl.ds(cnt, 16)], sv, mask=inm)
            plsc.store_compressed(cdst.at[pl.ds(cnt, 16)], d - lo, mask=inm)
            plsc.store_compressed(csp.at[pl.ds(cnt2, 16)], sv, mask=inm2)
            plsc.store_compressed(cdp.at[pl.ds(cnt2, 16)], d - lo2, mask=inm2)
            p1 = plsc.all_reduce_population_count(inm)
            p1 = p1 if p1.ndim == 0 else p1[0]
            p2 = plsc.all_reduce_population_count(inm2)
            p2 = p2 if p2.ndim == 0 else p2[0]
            return (cnt + p1, cnt2 + p2)
        cnt, cnt2 = lax.fori_loop(0, 64, _comp, (cnt, cnt2))

        nfull = lax.shift_right_logical(cnt, 7)
        _drain_pairs(nfull, csrc, cdst)

        @pl.when(nfull > 0)
        def _mv():
            def _mvv(j, _):
                csrc[pl.ds(j * 16, 16)] = csrc[pl.ds(nfull * 128 + j * 16, 16)]
                cdst[pl.ds(j * 16, 16)] = cdst[pl.ds(nfull * 128 + j * 16, 16)]
                return _
            lax.fori_loop(0, 8, _mvv, None)

        ng = lax.shift_right_logical(cnt2, 10)

        def _spill(f, spr):
            off = pl.multiple_of(f * 1024, 1024)
            pltpu.sync_copy(
                csp.at[pl.ds(off, 1024)],
                spsrc_hbm.at[c, s, pl.ds(spr * 1024, 1024)])
            pltpu.sync_copy(
                cdp.at[pl.ds(off, 1024)],
                spdl_hbm.at[c, s, pl.ds(spr * 1024, 1024)])
            return spr + 1
        spr = lax.fori_loop(0, ng, _spill, spr)

        @pl.when(ng > 0)
        def _mv2():
            def _mvv(j, _):
                csp[pl.ds(j * 16, 16)] = csp[pl.ds(ng * 1024 + j * 16, 16)]
                cdp[pl.ds(j * 16, 16)] = cdp[pl.ds(ng * 1024 + j * 16, 16)]
                return _
            lax.fori_loop(0, 64, _mvv, None)
        return (cnt & 127, cnt2 & 1023, spr)

    _stage(0, sblk, dblk)

    def _block2(b2, st):
        st = _do_block(2 * b2, st, sblk, dblk, sblk2, dblk2)
        st = _do_block(2 * b2 + 1, st, sblk2, dblk2, sblk, dblk)
        return st
    cnt, cnt2, spr = lax.fori_loop(
        0, nblk // 2, _block2,
        (jnp.int32(0), jnp.int32(0), jnp.int32(0)))

    @pl.when(cnt > 0)
    def _flush():
        def _pad(j, _):
            csrc[pl.ds(cnt + j * 16, 16)] = izeros
            cdst[pl.ds(cnt + j * 16, 16)] = idump
            return _
        lax.fori_loop(0, 8, _pad, None)
        hf = pltpu.async_copy(
            u2_hbm.at[csrc.at[pl.ds(0, 128)]],
            rowbuf.at[pl.ds(0, 128), :], sem)
        _fill_didx(cdst, 0, 0)
        hf.wait()
        pltpu.sync_copy(
            rowbuf.at[pl.ds(0, 128), :], acc.at[didx.at[0]], add=True)

    @pl.when(cnt2 > 0)
    def _flush2():
        def _pad(j, _):
            csp[pl.ds(cnt2 + j * 16, 16)] = izeros
            cdp[pl.ds(cnt2 + j * 16, 16)] = idump
            return _
        lax.fori_loop(0, 64, _pad, None)
        pltpu.sync_copy(
            csp.at[pl.ds(0, 1024)],
            spsrc_hbm.at[c, s, pl.ds(spr * 1024, 1024)])
        pltpu.sync_copy(
            cdp.at[pl.ds(0, 1024)],
            spdl_hbm.at[c, s, pl.ds(spr * 1024, 1024)])
    spr = spr + jnp.where(cnt2 > 0, 1, 0).astype(jnp.int32)

    plsc.subcore_barrier()
    pltpu.sync_copy(
        acc.at[pl.ds(s * (CHUNK // 16), CHUNK // 16), :],
        out_hbm.at[pl.ds(lo + s * (CHUNK // 16), CHUNK // 16), :])
    plsc.subcore_barrier()

    _zero_acc()
    plsc.subcore_barrier()

    def _gstage(g, ib, db):
        pltpu.async_copy(
            spsrc_hbm.at[c, s, pl.ds(g * 1024, 1024)],
            ib.at[pl.ds(0, 1024)], seme)
        pltpu.async_copy(
            spdl_hbm.at[c, s, pl.ds(g * 1024, 1024)],
            db.at[pl.ds(0, 1024)], seme)

    @pl.when(spr > 0)
    def _():
        _gstage(0, csp, cdp)

    def _grp(g, _):
        pltpu.make_async_copy(
            spsrc_hbm.at[c, s, pl.ds(g * 1024, 1024)],
            csp.at[pl.ds(0, 1024)], seme).wait()
        pltpu.make_async_copy(
            spdl_hbm.at[c, s, pl.ds(g * 1024, 1024)],
            cdp.at[pl.ds(0, 1024)], seme).wait()

        @pl.when((g & 1) == 0)
        def _even():
            @pl.when(g + 1 < spr)
            def _():
                _gstage(g + 1, csrc, cdst)
            _drain_pairs(jnp.int32(8), csp, cdp)

        @pl.when((g & 1) == 1)
        def _odd():
            @pl.when(g + 1 < spr)
            def _():
                _gstage(g + 1, csp, cdp)
            _drain_pairs(jnp.int32(8), csrc, cdst)
        return _
    lax.fori_loop(0, spr, _grp, None)

    plsc.subcore_barrier()
    pltpu.sync_copy(
        acc.at[pl.ds(s * (CHUNK // 16), CHUNK // 16), :],
        out_hbm.at[pl.ds(lo2 + s * (CHUNK // 16), CHUNK // 16), :])
    plsc.subcore_barrier()


def _stage_a0(dp_ref, deg_ref):
    deg_ref[...] = jnp.sum(dp_ref[...], axis=0)[:, None]


def _stage_a(deg_ref, x_ref, dis_ref, u16_ref):
    deg = deg_ref[...][:, 0] + 1.0
    dis = lax.rsqrt(deg)
    dis_ref[...] = dis[:, None]
    u16_ref[...] = jnp.concatenate(
        [dis[:, None] * x_ref[...], jnp.zeros((R, 14), _f32)], axis=1)


def _stage_b(t1_ref, u16_ref, dis_ref, w1_ref, b1_ref, w2_ref, u2_ref):
    t1 = t1_ref[0] + t1_ref[1]
    dis = dis_ref[...]
    s1 = dis * (t1[:, :2] + u16_ref[..., :2])
    h1 = jnp.maximum(
        jnp.dot(s1, w1_ref[...], preferred_element_type=_f32) + b1_ref[...],
        0.0)
    z = jnp.dot(h1, w2_ref[...], preferred_element_type=_f32)
    u2_ref[...] = dis * z


def _stage_c(t2_ref, u2_ref, dis_ref, b2_ref, wl_ref, bl_ref, out_ref, acc):
    i = pl.program_id(0)
    h2 = jnp.maximum(
        dis_ref[...] * (t2_ref[...] + u2_ref[...]) + b2_ref[...], 0.0)
    part = jnp.sum(h2, axis=0, keepdims=True)

    @pl.when(i == 0)
    def _():
        acc[...] = part

    @pl.when(i > 0)
    def _():
        acc[...] = acc[...] + part

    @pl.when(i == GRID - 1)
    def _():
        pooled = acc[...] / float(N_NODES)
        out_ref[...] = (
            jnp.dot(pooled, wl_ref[...], preferred_element_type=_f32)
            + bl_ref[...])


def kernel(x, edge_index, W1, b1, W2, b2, Wl, bl):
    ei = edge_index.astype(_i32)
    src = jnp.concatenate([ei[0], jnp.zeros((PAD_E,), _i32)])
    dst = jnp.concatenate([ei[1], jnp.full((PAD_E,), N_NODES, _i32)])
    src2d = src.reshape(EROWS, 128)
    dst2d = dst.reshape(EROWS, 128)

    degpart = _deg_sc(dst2d)

    deg2d = pl.pallas_call(
        _stage_a0,
        grid=(17,),
        in_specs=[pl.BlockSpec((32, 5888), lambda i: (0, i))],
        out_specs=pl.BlockSpec((5888, 1), lambda i: (i, 0)),
        out_shape=jax.ShapeDtypeStruct((N_PAD, 1), _f32),
    )(degpart)

    dis, u16 = pl.pallas_call(
        _stage_a,
        grid=(GRID,),
        in_specs=[
            pl.BlockSpec((R, 1), lambda i: (i, 0)),
            pl.BlockSpec((R, 2), lambda i: (i, 0)),
        ],
        out_specs=[
            pl.BlockSpec((R, 1), lambda i: (i, 0)),
            pl.BlockSpec((R, 16), lambda i: (i, 0)),
        ],
        out_shape=[
            jax.ShapeDtypeStruct((N_NODES, 1), _f32),
            jax.ShapeDtypeStruct((N_NODES, 16), _f32),
        ],
    )(deg2d, x)

    t1part = _seg1_sc(src2d, dst2d, u16)

    u2 = pl.pallas_call(
        _stage_b,
        grid=(GRID,),
        in_specs=[
            pl.BlockSpec((2, R, 16), lambda i: (0, i, 0)),
            pl.BlockSpec((R, 16), lambda i: (i, 0)),
            pl.BlockSpec((R, 1), lambda i: (i, 0)),
            pl.BlockSpec((2, 64), lambda i: (0, 0)),
            pl.BlockSpec((1, 64), lambda i: (0, 0)),
            pl.BlockSpec((64, 64), lambda i: (0, 0)),
        ],
        out_specs=pl.BlockSpec((R, 64), lambda i: (i, 0)),
        out_shape=jax.ShapeDtypeStruct((N_NODES, 64), _f32),
    )(t1part, u16, dis, W1, b1.reshape(1, 64), W2)

    t2, _sp1, _sp2 = _seg2_sc(src2d, dst2d, u2)

    out = pl.pallas_call(
        _stage_c,
        grid=(GRID,),
        in_specs=[
            pl.BlockSpec((R, 64), lambda i: (i, 0)),
            pl.BlockSpec((R, 64), lambda i: (i, 0)),
            pl.BlockSpec((R, 1), lambda i: (i, 0)),
            pl.BlockSpec((1, 64), lambda i: (0, 0)),
            pl.BlockSpec((64, 1), lambda i: (0, 0)),
            pl.BlockSpec((1, 1), lambda i: (0, 0)),
        ],
        out_specs=pl.BlockSpec((1, 1), lambda i: (0, 0)),
        out_shape=jax.ShapeDtypeStruct((1, 1), _f32),
        scratch_shapes=[pltpu.VMEM((1, 64), _f32)],
    )(t2, u2, dis, b2.reshape(1, 64), Wl, bl.reshape(1, 1))

    return out.reshape(1)


# speedup vs baseline: 1.2806x; 1.0654x over previous
import functools

import jax
import jax.numpy as jnp
from jax import lax
from jax.experimental import pallas as pl
from jax.experimental.pallas import tpu as pltpu
from jax.experimental.pallas import tpu_sc as plsc

N_NODES = 100000
N_EDGES = 6400000
N_PAD = 100096
E_PAD = 6422528
PAD_E = E_PAD - N_EDGES
EROWS = E_PAD // 128
CHUNK = 25600
ACC2 = CHUNK + 8
CAP = 1280
CAP2 = 2064
SPG = 400
R = 2000
GRID = N_NODES // R

_mesh = plsc.VectorSubcoreMesh(core_axis_name="c", subcore_axis_name="s")
_f32 = jnp.float32
_i32 = jnp.int32


@functools.partial(
    pl.kernel,
    out_type=jax.ShapeDtypeStruct((32, N_PAD), _f32),
    mesh=_mesh,
    compiler_params=pltpu.CompilerParams(needs_layout_passes=False),
    scratch_types=[
        pltpu.VMEM((N_PAD,), _f32),
        pltpu.VMEM((32, 128), _i32),
    ],
)
def _deg_sc(dst_hbm, out_hbm, acc, blk):
    c = lax.axis_index("c")
    s = lax.axis_index("s")
    wid = c * 16 + s
    zeros = jnp.zeros((16,), _f32)
    ones = jnp.ones((16,), _f32)

    def _zero(i, _):
        acc[pl.ds(i * 16, 16)] = zeros
        return _
    lax.fori_loop(0, N_PAD // 16, _zero, None)

    rows_per_tile = EROWS // 32
    base = wid * rows_per_tile

    def _block(b, _):
        pltpu.sync_copy(dst_hbm.at[pl.ds(base + b * 32, 32), :], blk)

        def _hist(j, _):
            g = j // 8
            k = j % 8
            idx = blk[g, pl.ds(k * 16, 16)]
            plsc.addupdate_scatter(acc, [idx], ones)
            return _
        lax.fori_loop(0, 32 * 8, _hist, None)
        return _
    lax.fori_loop(0, rows_per_tile // 32, _block, None)

    pltpu.sync_copy(acc, out_hbm.at[wid])


@functools.partial(
    pl.kernel,
    out_type=jax.ShapeDtypeStruct((2, N_PAD, 16), _f32),
    mesh=_mesh,
    compiler_params=pltpu.CompilerParams(
        needs_layout_passes=False, use_tc_tiling_on_sc=False),
    scratch_types=[
        pltpu.VMEM_SHARED((N_PAD, 16), _f32),
        pltpu.VMEM((8, 128), _i32),
        pltpu.VMEM((8, 128), _i32),
        pltpu.VMEM((8, 128), _i32),
        pltpu.VMEM((8, 128), _i32),
        pltpu.VMEM((1024, 16), _f32),
        pltpu.SemaphoreType.DMA,
        pltpu.SemaphoreType.DMA,
        pltpu.SemaphoreType.DMA,
    ],
)
def _seg1_sc(src_hbm, dst_hbm, u16_hbm, out_hbm, acc, sblk, dblk, sblk2,
             dblk2, rows, sem, sem2, seme):
    c = lax.axis_index("c")
    s = lax.axis_index("s")
    zeros = jnp.zeros((16,), _f32)

    def _zb(i, _):
        rows[i, :] = zeros
        return _
    lax.fori_loop(0, 782, _zb, None)

    zr = s * (N_PAD // 16)

    def _za(k, _):
        pltpu.sync_copy(
            rows.at[pl.ds(0, 782), :], acc.at[pl.ds(zr + k * 782, 782), :])
        return _
    lax.fori_loop(0, 8, _za, None)
    plsc.subcore_barrier()

    rows_per_tile = EROWS // 32
    nblk = rows_per_tile // 8
    base = c * (EROWS // 2) + s * rows_per_tile

    def _stage(b, sb, db):
        r0 = base + b * 8
        pltpu.async_copy(src_hbm.at[pl.ds(r0, 8), :], sb, seme)
        pltpu.async_copy(dst_hbm.at[pl.ds(r0, 8), :], db, seme)

    def _do_block(b, sb, db, sbn, dbn):
        r0 = base + b * 8
        pltpu.make_async_copy(src_hbm.at[pl.ds(r0, 8), :], sb, seme).wait()
        pltpu.make_async_copy(dst_hbm.at[pl.ds(r0, 8), :], db, seme).wait()

        @pl.when(b + 1 < nblk)
        def _():
            _stage(b + 1, sbn, dbn)
        hs = [
            pltpu.async_copy(
                u16_hbm.at[sb.at[g]], rows.at[pl.ds(g * 128, 128), :], sem)
            for g in range(8)
        ]
        for h in hs:
            h.wait()
        ws = [
            pltpu.async_copy(
                rows.at[pl.ds(g * 128, 128), :], acc.at[db.at[g]], sem2,
                add=True)
            for g in range(8)
        ]
        for w in ws:
            w.wait()

    _stage(0, sblk, dblk)

    def _block2(b2, _):
        _do_block(2 * b2, sblk, dblk, sblk2, dblk2)
        _do_block(2 * b2 + 1, sblk2, dblk2, sblk, dblk)
        return _
    lax.fori_loop(0, nblk // 2, _block2, None)
    plsc.subcore_barrier()

    pltpu.sync_copy(
        acc.at[pl.ds(zr, N_PAD // 16), :],
        out_hbm.at[c, pl.ds(zr, N_PAD // 16), :])


@functools.partial(
    pl.kernel,
    out_type=[
        jax.ShapeDtypeStruct((4 * CHUNK, 64), _f32),
        jax.ShapeDtypeStruct((2, 16, SPG * 1024), _i32),
        jax.ShapeDtypeStruct((2, 16, SPG * 1024), _i32),
    ],
    mesh=_mesh,
    compiler_params=pltpu.CompilerParams(
        needs_layout_passes=False, use_tc_tiling_on_sc=False),
    scratch_types=[
        pltpu.VMEM_SHARED((ACC2, 64), _f32),
        pltpu.VMEM((8, 128), _i32),
        pltpu.VMEM((8, 128), _i32),
        pltpu.VMEM((8, 128), _i32),
        pltpu.VMEM((8, 128), _i32),
        pltpu.VMEM((CAP,), _i32),
        pltpu.VMEM((CAP,), _i32),
        pltpu.VMEM((CAP2,), _i32),
        pltpu.VMEM((CAP2,), _i32),
        pltpu.VMEM((2, 128), _i32),
        pltpu.VMEM((256, 64), _f32),
        pltpu.SemaphoreType.DMA,
        pltpu.SemaphoreType.DMA,
        pltpu.SemaphoreType.DMA,
        pltpu.SemaphoreType.DMA,
        pltpu.SemaphoreType.DMA,
        pltpu.SemaphoreType.DMA,
    ],
)
def _seg2_sc(src_hbm, dst_hbm, u2_hbm, out_hbm, spsrc_hbm, spdl_hbm,
             acc, sblk, dblk, sblk2, dblk2, csrc, cdst, csp, cdp, didx,
             rowbuf, sem, seme, sg0, sg1, ss0, ss1):
    c = lax.axis_index("c")
    s = lax.axis_index("s")
    zeros = jnp.zeros((16,), _f32)
    izeros = jnp.zeros((16,), _i32)
    idump = jnp.full((16,), CHUNK, _i32)

    rows_per_tile = EROWS // 16
    base = s * rows_per_tile
    lo = c * CHUNK
    lo2 = (c + 2) * CHUNK

    def _zero_acc():
        def _zb(i, _):
            g = i // 4
            k = i % 4
            rowbuf[g, pl.ds(k * 16, 16)] = zeros
            return _
        lax.fori_loop(0, 64 * 4, _zb, None)

        def _za(k, _):
            pltpu.sync_copy(
                rowbuf.at[pl.ds(0, 64), :],
                acc.at[pl.ds(s * (CHUNK // 16) + k * 64, 64), :])
            return _
        lax.fori_loop(0, 25, _za, None)

    def _fill_didx(dlbuf, off, slot):
        def _fd(j, _):
            didx[slot, pl.ds(j * 16, 16)] = dlbuf[pl.ds(off + j * 16, 16)]
            return _
        lax.fori_loop(0, 8, _fd, None)

    def _drain_pairs(nfull, idxbuf, dlbuf):
        halves = (rowbuf.at[pl.ds(0, 128), :], rowbuf.at[pl.ds(128, 128), :])
        gsems = (sg0, sg1)
        ssems = (ss0, ss1)

        def _fire_gather(r, h):
            off = pl.multiple_of(r * 128, 128)
            pltpu.async_copy(
                u2_hbm.at[idxbuf.at[pl.ds(off, 128)]], halves[h], gsems[h])

        @pl.when(nfull > 0)
        def _():
            _fire_gather(0, 0)

        def _one(r, h):
            ho = 1 - h

            @pl.when((r + 1 < nfull) & (r >= 1))
            def _():
                pltpu.make_async_copy(
                    u2_hbm.at[pl.ds(0, 128), :], halves[ho], ssems[ho]).wait()

            @pl.when(r + 1 < nfull)
            def _():
                _fire_gather(r + 1, ho)
            _fill_didx(dlbuf, pl.multiple_of(r * 128, 128), h)
            pltpu.make_async_copy(
                u2_hbm.at[pl.ds(0, 128), :], halves[h], gsems[h]).wait()
            pltpu.async_copy(halves[h], acc.at[didx.at[h]], ssems[h], add=True)

        def _step(r, _):
            @pl.when((r & 1) == 0)
            def _():
                _one(r, 0)

            @pl.when((r & 1) == 1)
            def _():
                _one(r, 1)
            return _
        lax.fori_loop(0, nfull, _step, None)

        @pl.when(nfull >= 1)
        def _():
            pltpu.make_async_copy(
                u2_hbm.at[pl.ds(0, 128), :], halves[0], ssems[0]).wait()

        @pl.when(nfull >= 2)
        def _():
            pltpu.make_async_copy(
                u2_hbm.at[pl.ds(0, 128), :], halves[1], ssems[1]).wait()

    _zero_acc()
    plsc.subcore_barrier()

    nblk = rows_per_tile // 8

    def _stage(b, sb, db):
        r0 = base + b * 8
        pltpu.async_copy(src_hbm.at[pl.ds(r0, 8), :], sb, seme)
        pltpu.async_copy(dst_hbm.at[pl.ds(r0, 8), :], db, seme)

    def _do_block(b, st, sb, db, sbn, dbn):
        cnt, cnt2, spr = st
        r0 = base + b * 8
        pltpu.make_async_copy(src_hbm.at[pl.ds(r0, 8), :], sb, seme).wait()
        pltpu.make_async_copy(dst_hbm.at[pl.ds(r0, 8), :], db, seme).wait()

        @pl.when(b + 1 < nblk)
        def _():
            _stage(b + 1, sbn, dbn)

        def _comp(j, st2):
            cnt, cnt2 = st2
            g = j // 8
            k = j % 8
            d = db[g, pl.ds(k * 16, 16)]
            sv = sb[g, pl.ds(k * 16, 16)]
            inm = (d >= lo) & (d < lo + CHUNK)
            inm2 = (d >= lo2) & (d < lo2 + CHUNK)
            plsc.store_compressed(csrc.at[pl.ds(cnt, 16)], sv, mask=inm)
            plsc.store_compressed(cdst.at[pl.ds(cnt, 16)], d - lo, mask=inm)
            plsc.store_compressed(csp.at[pl.ds(cnt2, 16)], sv, mask=inm2)
            plsc.store_compressed(cdp.at[pl.ds(cnt2, 16)], d - lo2, mask=inm2)
            p1 = plsc.all_reduce_population_count(inm)
            p1 = p1 if p1.ndim == 0 else p1[0]
            p2 = plsc.all_reduce_population_count(inm2)
            p2 = p2 if p2.ndim == 0 else p2[0]
            return (cnt + p1, cnt2 + p2)
        cnt, cnt2 = lax.fori_loop(0, 64, _comp, (cnt, cnt2))

        nfull = lax.shift_right_logical(cnt, 7)
        _drain_pairs(nfull, csrc, cdst)

        @pl.when(nfull > 0)
        def _mv():
            def _mvv(j, _):
                csrc[pl.ds(j * 16, 16)] = csrc[pl.ds(nfull * 128 + j * 16, 16)]
                cdst[pl.ds(j * 16, 16)] = cdst[pl.ds(nfull * 128 + j * 16, 16)]
                return _
            lax.fori_loop(0, 8, _mvv, None)

        ng = lax.shift_right_logical(cnt2, 10)

        def _spill(f, spr):
            off = pl.multiple_of(f * 1024, 1024)
            pltpu.sync_copy(
                csp.at[pl.ds(off, 1024)],
                spsrc_hbm.at[c, s, pl.ds(spr * 1024, 1024)])
            pltpu.sync_copy(
                cdp.at[pl.ds(off, 1024)],
                spdl_hbm.at[c, s, pl.ds(spr * 1024, 1024)])
            return spr + 1
        spr = lax.fori_loop(0, ng, _spill, spr)

        @pl.when(ng > 0)
        def _mv2():
            def _mvv(j, _):
                csp[pl.ds(j * 16, 16)] = csp[pl.ds(ng * 1024 + j * 16, 16)]
                cdp[pl.ds(j * 16, 16)] = cdp[pl.ds(ng * 1024 + j * 16, 16)]
                return _
            lax.fori_loop(0, 64, _mvv, None)
        return (cnt & 127, cnt2 & 1023, spr)

    _stage(0, sblk, dblk)

    def _block2(b2, st):
        st = _do_block(2 * b2, st, sblk, dblk, sblk2, dblk2)
        st = _do_block(2 * b2 + 1, st, sblk2, dblk2, sblk, dblk)
        return st
    cnt, cnt2, spr = lax.fori_loop(
        0, nblk // 2, _block2,
        (jnp.int32(0), jnp.int32(0), jnp.int32(0)))

    @pl.when(cnt > 0)
    def _flush():
        def _pad(j, _):
            csrc[pl.ds(cnt + j * 16, 16)] = izeros
            cdst[pl.ds(cnt + j * 16, 16)] = idump
            return _
        lax.fori_loop(0, 8, _pad, None)
        hf = pltpu.async_copy(
            u2_hbm.at[csrc.at[pl.ds(0, 128)]],
            rowbuf.at[pl.ds(0, 128), :], sem)
        _fill_didx(cdst, 0, 0)
        hf.wait()
        pltpu.sync_copy(
            rowbuf.at[pl.ds(0, 128), :], acc.at[didx.at[0]], add=True)

    @pl.when(cnt2 > 0)
    def _flush2():
        def _pad(j, _):
            csp[pl.ds(cnt2 + j * 16, 16)] = izeros
            cdp[pl.ds(cnt2 + j * 16, 16)] = idump
            return _
        lax.fori_loop(0, 64, _pad, None)
        pltpu.sync_copy(
            csp.at[pl.ds(0, 1024)],
            spsrc_hbm.at[c, s, pl.ds(spr * 1024, 1024)])
        pltpu.sync_copy(
            cdp.at[pl.ds(0, 1024)],
            spdl_hbm.at[c, s, pl.ds(spr * 1024, 1024)])
    spr = spr + jnp.where(cnt2 > 0, 1, 0).astype(jnp.int32)

    plsc.subcore_barrier()
    pltpu.sync_copy(
        acc.at[pl.ds(s * (CHUNK // 16), CHUNK // 16), :],
        out_hbm.at[pl.ds(lo + s * (CHUNK // 16), CHUNK // 16), :])
    plsc.subcore_barrier()

    _zero_acc()
    plsc.subcore_barrier()

    def _gstage(g, ib, db):
        pltpu.async_copy(
            spsrc_hbm.at[c, s, pl.ds(g * 1024, 1024)],
            ib.at[pl.ds(0, 1024)], seme)
        pltpu.async_copy(
            spdl_hbm.at[c, s, pl.ds(g * 1024, 1024)],
            db.at[pl.ds(0, 1024)], seme)

    @pl.when(spr > 0)
    def _():
        _gstage(0, csp, cdp)

    def _grp(g, _):
        pltpu.make_async_copy(
            spsrc_hbm.at[c, s, pl.ds(g * 1024, 1024)],
            csp.at[pl.ds(0, 1024)], seme).wait()
        pltpu.make_async_copy(
            spdl_hbm.at[c, s, pl.ds(g * 1024, 1024)],
            cdp.at[pl.ds(0, 1024)], seme).wait()

        @pl.when((g & 1) == 0)
        def _even():
            @pl.when(g + 1 < spr)
            def _():
                _gstage(g + 1, csrc, cdst)
            _drain_pairs(jnp.int32(8), csp, cdp)

        @pl.when((g & 1) == 1)
        def _odd():
            @pl.when(g + 1 < spr)
            def _():
                _gstage(g + 1, csp, cdp)
            _drain_pairs(jnp.int32(8), csrc, cdst)
        return _
    lax.fori_loop(0, spr, _grp, None)

    plsc.subcore_barrier()
    pltpu.sync_copy(
        acc.at[pl.ds(s * (CHUNK // 16), CHUNK // 16), :],
        out_hbm.at[pl.ds(lo2 + s * (CHUNK // 16), CHUNK // 16), :])
    plsc.subcore_barrier()


def _stage_a0(dp_ref, deg_ref):
    deg_ref[...] = jnp.sum(dp_ref[...], axis=0)[:, None]


def _stage_a(deg_ref, x_ref, dis_ref, u16_ref):
    deg = deg_ref[...][:, 0] + 1.0
    dis = lax.rsqrt(deg)
    dis_ref[...] = dis[:, None]
    u16_ref[...] = jnp.concatenate(
        [dis[:, None] * x_ref[...], jnp.zeros((R, 14), _f32)], axis=1)


def _stage_b(t1_ref, u16_ref, dis_ref, w1_ref, b1_ref, w2_ref, u2_ref):
    t1 = t1_ref[0] + t1_ref[1]
    dis = dis_ref[...]
    s1 = dis * (t1[:, :2] + u16_ref[..., :2])
    h1 = jnp.maximum(
        jnp.dot(s1, w1_ref[...], preferred_element_type=_f32) + b1_ref[...],
        0.0)
    z = jnp.dot(h1, w2_ref[...], preferred_element_type=_f32)
    u2_ref[...] = dis * z


def _stage_c(t2_ref, u2_ref, dis_ref, b2_ref, wl_ref, bl_ref, out_ref, acc):
    i = pl.program_id(0)
    h2 = jnp.maximum(
        dis_ref[...] * (t2_ref[...] + u2_ref[...]) + b2_ref[...], 0.0)
    part = jnp.sum(h2, axis=0, keepdims=True)

    @pl.when(i == 0)
    def _():
        acc[...] = part

    @pl.when(i > 0)
    def _():
        acc[...] = acc[...] + part

    @pl.when(i == GRID - 1)
    def _():
        pooled = acc[...] / float(N_NODES)
        out_ref[...] = (
            jnp.dot(pooled, wl_ref[...], preferred_element_type=_f32)
            + bl_ref[...])


def kernel(x, edge_index, W1, b1, W2, b2, Wl, bl):
    ei = edge_index.astype(_i32)
    src = jnp.concatenate([ei[0], jnp.zeros((PAD_E,), _i32)])
    dst = jnp.concatenate([ei[1], jnp.full((PAD_E,), N_NODES, _i32)])
    src2d = src.reshape(EROWS, 128)
    dst2d = dst.reshape(EROWS, 128)

    degpart = _deg_sc(dst2d)

    deg2d = pl.pallas_call(
        _stage_a0,
        grid=(17,),
        in_specs=[pl.BlockSpec((32, 5888), lambda i: (0, i))],
        out_specs=pl.BlockSpec((5888, 1), lambda i: (i, 0)),
        out_shape=jax.ShapeDtypeStruct((N_PAD, 1), _f32),
    )(degpart)

    dis, u16 = pl.pallas_call(
        _stage_a,
        grid=(GRID,),
        in_specs=[
            pl.BlockSpec((R, 1), lambda i: (i, 0)),
            pl.BlockSpec((R, 2), lambda i: (i, 0)),
        ],
        out_specs=[
            pl.BlockSpec((R, 1), lambda i: (i, 0)),
            pl.BlockSpec((R, 16), lambda i: (i, 0)),
        ],
        out_shape=[
            jax.ShapeDtypeStruct((N_NODES, 1), _f32),
            jax.ShapeDtypeStruct((N_NODES, 16), _f32),
        ],
    )(deg2d, x)

    t1part = _seg1_sc(src2d, dst2d, u16)

    u2 = pl.pallas_call(
        _stage_b,
        grid=(GRID,),
        in_specs=[
            pl.BlockSpec((2, R, 16), lambda i: (0, i, 0)),
            pl.BlockSpec((R, 16), lambda i: (i, 0)),
            pl.BlockSpec((R, 1), lambda i: (i, 0)),
            pl.BlockSpec((2, 64), lambda i: (0, 0)),
            pl.BlockSpec((1, 64), lambda i: (0, 0)),
            pl.BlockSpec((64, 64), lambda i: (0, 0)),
        ],
        out_specs=pl.BlockSpec((R, 64), lambda i: (i, 0)),
        out_shape=jax.ShapeDtypeStruct((N_NODES, 64), _f32),
    )(t1part, u16, dis, W1, b1.reshape(1, 64), W2)

    t2, _sp1, _sp2 = _seg2_sc(src2d, dst2d, u2)

    out = pl.pallas_call(
        _stage_c,
        grid=(GRID,),
        in_specs=[
            pl.BlockSpec((R, 64), lambda i: (i, 0)),
            pl.BlockSpec((R, 64), lambda i: (i, 0)),
            pl.BlockSpec((R, 1), lambda i: (i, 0)),
            pl.BlockSpec((1, 64), lambda i: (0, 0)),
            pl.BlockSpec((64, 1), lambda i: (0, 0)),
            pl.BlockSpec((1, 1), lambda i: (0, 0)),
        ],
        out_specs=pl.BlockSpec((1, 1), lambda i: (0, 0)),
        out_shape=jax.ShapeDtypeStruct((1, 1), _f32),
        scratch_shapes=[pltpu.VMEM((1, 64), _f32)],
    )(t2, u2, dis, b2.reshape(1, 64), Wl, bl.reshape(1, 1))

    return out.reshape(1)


# speedup vs baseline: 1.2842x; 1.0028x over previous
import functools

import jax
import jax.numpy as jnp
from jax import lax
from jax.experimental import pallas as pl
from jax.experimental.pallas import tpu as pltpu
from jax.experimental.pallas import tpu_sc as plsc

N_NODES = 100000
N_EDGES = 6400000
N_PAD = 100096
E_PAD = 6422528
PAD_E = E_PAD - N_EDGES
EROWS = E_PAD // 128
CHUNK = 25600
ACC2 = CHUNK + 8
CAP = 1280
CAP2 = 2064
SPG = 400
R = 2000
GRID = N_NODES // R

_mesh = plsc.VectorSubcoreMesh(core_axis_name="c", subcore_axis_name="s")
_f32 = jnp.float32
_i32 = jnp.int32


@functools.partial(
    pl.kernel,
    out_type=jax.ShapeDtypeStruct((32, N_PAD), _f32),
    mesh=_mesh,
    compiler_params=pltpu.CompilerParams(needs_layout_passes=False),
    scratch_types=[
        pltpu.VMEM((N_PAD,), _f32),
        pltpu.VMEM((32, 128), _i32),
    ],
)
def _deg_sc(dst_hbm, out_hbm, acc, blk):
    c = lax.axis_index("c")
    s = lax.axis_index("s")
    wid = c * 16 + s
    zeros = jnp.zeros((16,), _f32)
    ones = jnp.ones((16,), _f32)

    def _zero(i, _):
        acc[pl.ds(i * 16, 16)] = zeros
        return _
    lax.fori_loop(0, N_PAD // 16, _zero, None)

    rows_per_tile = EROWS // 32
    base = wid * rows_per_tile

    def _block(b, _):
        pltpu.sync_copy(dst_hbm.at[pl.ds(base + b * 32, 32), :], blk)

        def _hist(j, _):
            g = j // 8
            k = j % 8
            idx = blk[g, pl.ds(k * 16, 16)]
            plsc.addupdate_scatter(acc, [idx], ones)
            return _
        lax.fori_loop(0, 32 * 8, _hist, None)
        return _
    lax.fori_loop(0, rows_per_tile // 32, _block, None)

    pltpu.sync_copy(acc, out_hbm.at[wid])


@functools.partial(
    pl.kernel,
    out_type=jax.ShapeDtypeStruct((2, N_PAD, 16), _f32),
    mesh=_mesh,
    compiler_params=pltpu.CompilerParams(
        needs_layout_passes=False, use_tc_tiling_on_sc=False),
    scratch_types=[
        pltpu.VMEM_SHARED((N_PAD, 16), _f32),
        pltpu.VMEM((4, 128), _i32),
        pltpu.VMEM((4, 128), _i32),
        pltpu.VMEM((4, 128), _i32),
        pltpu.VMEM((4, 128), _i32),
        pltpu.VMEM((512, 16), _f32),
        pltpu.VMEM((512, 16), _f32),
        pltpu.SemaphoreType.DMA,
        pltpu.SemaphoreType.DMA,
        pltpu.SemaphoreType.DMA,
        pltpu.SemaphoreType.DMA,
    ],
)
def _seg1_sc(src_hbm, dst_hbm, u16_hbm, out_hbm, acc, sblk, dblk, sblk2,
             dblk2, rows, rows2, sem, ssa, ssb, seme):
    c = lax.axis_index("c")
    s = lax.axis_index("s")
    zeros = jnp.zeros((16,), _f32)

    def _zb(i, _):
        rows[i, :] = zeros
        return _
    lax.fori_loop(0, 512, _zb, None)

    zr = s * (N_PAD // 16)

    def _za(k, _):
        pltpu.sync_copy(
            rows, acc.at[pl.ds(zr + k * 512, 512), :])
        return _
    lax.fori_loop(0, 12, _za, None)
    pltpu.sync_copy(
        rows.at[pl.ds(0, 112), :], acc.at[pl.ds(zr + 6144, 112), :])
    plsc.subcore_barrier()

    rows_per_tile = EROWS // 32
    nblk = rows_per_tile // 4
    base = c * (EROWS // 2) + s * rows_per_tile

    def _stage(b, sb, db):
        r0 = base + b * 4
        pltpu.async_copy(src_hbm.at[pl.ds(r0, 4), :], sb, seme)
        pltpu.async_copy(dst_hbm.at[pl.ds(r0, 4), :], db, seme)

    def _do_block(b, sb, db, sbn, dbn, rb, ssem, first):
        r0 = base + b * 4
        pltpu.make_async_copy(src_hbm.at[pl.ds(r0, 4), :], sb, seme).wait()
        pltpu.make_async_copy(dst_hbm.at[pl.ds(r0, 4), :], db, seme).wait()

        @pl.when(b + 1 < nblk)
        def _():
            _stage(b + 1, sbn, dbn)
        if not first:
            for g in range(4):
                pltpu.make_async_copy(
                    u16_hbm.at[pl.ds(0, 128)],
                    rb.at[pl.ds(g * 128, 128), :], ssem).wait()
        hs = [
            pltpu.async_copy(
                u16_hbm.at[sb.at[g]], rb.at[pl.ds(g * 128, 128), :], sem)
            for g in range(4)
        ]
        for h in hs:
            h.wait()
        for g in range(4):
            pltpu.async_copy(
                rb.at[pl.ds(g * 128, 128), :], acc.at[db.at[g]], ssem,
                add=True)

    _stage(0, sblk, dblk)
    _do_block(0, sblk, dblk, sblk2, dblk2, rows, ssa, True)
    _do_block(1, sblk2, dblk2, sblk, dblk, rows2, ssb, True)

    def _block2(b2, _):
        _do_block(2 * b2, sblk, dblk, sblk2, dblk2, rows, ssa, False)
        _do_block(2 * b2 + 1, sblk2, dblk2, sblk, dblk, rows2, ssb, False)
        return _
    lax.fori_loop(1, nblk // 2, _block2, None)
    for g in range(4):
        pltpu.make_async_copy(
            u16_hbm.at[pl.ds(0, 128)],
            rows.at[pl.ds(g * 128, 128), :], ssa).wait()
        pltpu.make_async_copy(
            u16_hbm.at[pl.ds(0, 128)],
            rows2.at[pl.ds(g * 128, 128), :], ssb).wait()
    plsc.subcore_barrier()

    pltpu.sync_copy(
        acc.at[pl.ds(zr, N_PAD // 16), :],
        out_hbm.at[c, pl.ds(zr, N_PAD // 16), :])


@functools.partial(
    pl.kernel,
    out_type=[
        jax.ShapeDtypeStruct((4 * CHUNK, 64), _f32),
        jax.ShapeDtypeStruct((2, 16, SPG * 1024), _i32),
        jax.ShapeDtypeStruct((2, 16, SPG * 1024), _i32),
    ],
    mesh=_mesh,
    compiler_params=pltpu.CompilerParams(
        needs_layout_passes=False, use_tc_tiling_on_sc=False),
    scratch_types=[
        pltpu.VMEM_SHARED((ACC2, 64), _f32),
        pltpu.VMEM((8, 128), _i32),
        pltpu.VMEM((8, 128), _i32),
        pltpu.VMEM((8, 128), _i32),
        pltpu.VMEM((8, 128), _i32),
        pltpu.VMEM((CAP,), _i32),
        pltpu.VMEM((CAP,), _i32),
        pltpu.VMEM((CAP2,), _i32),
        pltpu.VMEM((CAP2,), _i32),
        pltpu.VMEM((2, 128), _i32),
        pltpu.VMEM((256, 64), _f32),
        pltpu.SemaphoreType.DMA,
        pltpu.SemaphoreType.DMA,
        pltpu.SemaphoreType.DMA,
        pltpu.SemaphoreType.DMA,
        pltpu.SemaphoreType.DMA,
        pltpu.SemaphoreType.DMA,
    ],
)
def _seg2_sc(src_hbm, dst_hbm, u2_hbm, out_hbm, spsrc_hbm, spdl_hbm,
             acc, sblk, dblk, sblk2, dblk2, csrc, cdst, csp, cdp, didx,
             rowbuf, sem, seme, sg0, sg1, ss0, ss1):
    c = lax.axis_index("c")
    s = lax.axis_index("s")
    zeros = jnp.zeros((16,), _f32)
    izeros = jnp.zeros((16,), _i32)
    idump = jnp.full((16,), CHUNK, _i32)

    rows_per_tile = EROWS // 16
    base = s * rows_per_tile
    lo = c * CHUNK
    lo2 = (c + 2) * CHUNK

    def _zero_acc():
        def _zb(i, _):
            g = i // 4
            k = i % 4
            rowbuf[g, pl.ds(k * 16, 16)] = zeros
            return _
        lax.fori_loop(0, 64 * 4, _zb, None)

        def _za(k, _):
            pltpu.sync_copy(
                rowbuf.at[pl.ds(0, 64), :],
                acc.at[pl.ds(s * (CHUNK // 16) + k * 64, 64), :])
            return _
        lax.fori_loop(0, 25, _za, None)

    def _fill_didx(dlbuf, off, slot):
        def _fd(j, _):
            didx[slot, pl.ds(j * 16, 16)] = dlbuf[pl.ds(off + j * 16, 16)]
            return _
        lax.fori_loop(0, 8, _fd, None)

    def _drain_pairs(nfull, idxbuf, dlbuf):
        halves = (rowbuf.at[pl.ds(0, 128), :], rowbuf.at[pl.ds(128, 128), :])
        gsems = (sg0, sg1)
        ssems = (ss0, ss1)

        def _fire_gather(r, h):
            off = pl.multiple_of(r * 128, 128)
            pltpu.async_copy(
                u2_hbm.at[idxbuf.at[pl.ds(off, 128)]], halves[h], gsems[h])

        @pl.when(nfull > 0)
        def _():
            _fire_gather(0, 0)

        def _one(r, h):
            ho = 1 - h

            @pl.when((r + 1 < nfull) & (r >= 1))
            def _():
                pltpu.make_async_copy(
                    u2_hbm.at[pl.ds(0, 128), :], halves[ho], ssems[ho]).wait()

            @pl.when(r + 1 < nfull)
            def _():
                _fire_gather(r + 1, ho)
            _fill_didx(dlbuf, pl.multiple_of(r * 128, 128), h)
            pltpu.make_async_copy(
                u2_hbm.at[pl.ds(0, 128), :], halves[h], gsems[h]).wait()
            pltpu.async_copy(halves[h], acc.at[didx.at[h]], ssems[h], add=True)

        def _step(r, _):
            @pl.when((r & 1) == 0)
            def _():
                _one(r, 0)

            @pl.when((r & 1) == 1)
            def _():
                _one(r, 1)
            return _
        lax.fori_loop(0, nfull, _step, None)

        @pl.when(nfull >= 1)
        def _():
            pltpu.make_async_copy(
                u2_hbm.at[pl.ds(0, 128), :], halves[0], ssems[0]).wait()

        @pl.when(nfull >= 2)
        def _():
            pltpu.make_async_copy(
                u2_hbm.at[pl.ds(0, 128), :], halves[1], ssems[1]).wait()

    _zero_acc()
    plsc.subcore_barrier()

    nblk = rows_per_tile // 8

    def _stage(b, sb, db):
        r0 = base + b * 8
        pltpu.async_copy(src_hbm.at[pl.ds(r0, 8), :], sb, seme)
        pltpu.async_copy(dst_hbm.at[pl.ds(r0, 8), :], db, seme)

    def _do_block(b, st, sb, db, sbn, dbn):
        cnt, cnt2, spr = st
        r0 = base + b * 8
        pltpu.make_async_copy(src_hbm.at[pl.ds(r0, 8), :], sb, seme).wait()
        pltpu.make_async_copy(dst_hbm.at[pl.ds(r0, 8), :], db, seme).wait()

        @pl.when(b + 1 < nblk)
        def _():
            _stage(b + 1, sbn, dbn)

        def _comp(j, st2):
            cnt, cnt2 = st2
            g = j // 8
            k = j % 8
            d = db[g, pl.ds(k * 16, 16)]
            sv = sb[g, pl.ds(k * 16, 16)]
            inm = (d >= lo) & (d < lo + CHUNK)
            inm2 = (d >= lo2) & (d < lo2 + CHUNK)
            plsc.store_compressed(csrc.at[pl.ds(cnt, 16)], sv, mask=inm)
            plsc.store_compressed(cdst.at[pl.ds(cnt, 16)], d - lo, mask=inm)
            plsc.store_compressed(csp.at[pl.ds(cnt2, 16)], sv, mask=inm2)
            plsc.store_compressed(cdp.at[pl.ds(cnt2, 16)], d - lo2, mask=inm2)
            p1 = plsc.all_reduce_population_count(inm)
            p1 = p1 if p1.ndim == 0 else p1[0]
            p2 = plsc.all_reduce_population_count(inm2)
            p2 = p2 if p2.ndim == 0 else p2[0]
            return (cnt + p1, cnt2 + p2)
        cnt, cnt2 = lax.fori_loop(0, 64, _comp, (cnt, cnt2))

        nfull = lax.shift_right_logical(cnt, 7)
        _drain_pairs(nfull, csrc, cdst)

        @pl.when(nfull > 0)
        def _mv():
            def _mvv(j, _):
                csrc[pl.ds(j * 16, 16)] = csrc[pl.ds(nfull * 128 + j * 16, 16)]
                cdst[pl.ds(j * 16, 16)] = cdst[pl.ds(nfull * 128 + j * 16, 16)]
                return _
            lax.fori_loop(0, 8, _mvv, None)

        ng = lax.shift_right_logical(cnt2, 10)

        def _spill(f, spr):
            off = pl.multiple_of(f * 1024, 1024)
            pltpu.sync_copy(
                csp.at[pl.ds(off, 1024)],
                spsrc_hbm.at[c, s, pl.ds(spr * 1024, 1024)])
            pltpu.sync_copy(
                cdp.at[pl.ds(off, 1024)],
                spdl_hbm.at[c, s, pl.ds(spr * 1024, 1024)])
            return spr + 1
        spr = lax.fori_loop(0, ng, _spill, spr)

        @pl.when(ng > 0)
        def _mv2():
            def _mvv(j, _):
                csp[pl.ds(j * 16, 16)] = csp[pl.ds(ng * 1024 + j * 16, 16)]
                cdp[pl.ds(j * 16, 16)] = cdp[pl.ds(ng * 1024 + j * 16, 16)]
                return _
            lax.fori_loop(0, 64, _mvv, None)
        return (cnt & 127, cnt2 & 1023, spr)

    _stage(0, sblk, dblk)

    def _block2(b2, st):
        st = _do_block(2 * b2, st, sblk, dblk, sblk2, dblk2)
        st = _do_block(2 * b2 + 1, st, sblk2, dblk2, sblk, dblk)
        return st
    cnt, cnt2, spr = lax.fori_loop(
        0, nblk // 2, _block2,
        (jnp.int32(0), jnp.int32(0), jnp.int32(0)))

    @pl.when(cnt > 0)
    def _flush():
        def _pad(j, _):
            csrc[pl.ds(cnt + j * 16, 16)] = izeros
            cdst[pl.ds(cnt + j * 16, 16)] = idump
            return _
        lax.fori_loop(0, 8, _pad, None)
        hf = pltpu.async_copy(
            u2_hbm.at[csrc.at[pl.ds(0, 128)]],
            rowbuf.at[pl.ds(0, 128), :], sem)
        _fill_didx(cdst, 0, 0)
        hf.wait()
        pltpu.sync_copy(
            rowbuf.at[pl.ds(0, 128), :], acc.at[didx.at[0]], add=True)

    @pl.when(cnt2 > 0)
    def _flush2():
        def _pad(j, _):
            csp[pl.ds(cnt2 + j * 16, 16)] = izeros
            cdp[pl.ds(cnt2 + j * 16, 16)] = idump
            return _
        lax.fori_loop(0, 64, _pad, None)
        pltpu.sync_copy(
            csp.at[pl.ds(0, 1024)],
            spsrc_hbm.at[c, s, pl.ds(spr * 1024, 1024)])
        pltpu.sync_copy(
            cdp.at[pl.ds(0, 1024)],
            spdl_hbm.at[c, s, pl.ds(spr * 1024, 1024)])
    spr = spr + jnp.where(cnt2 > 0, 1, 0).astype(jnp.int32)

    plsc.subcore_barrier()
    pltpu.sync_copy(
        acc.at[pl.ds(s * (CHUNK // 16), CHUNK // 16), :],
        out_hbm.at[pl.ds(lo + s * (CHUNK // 16), CHUNK // 16), :])
    plsc.subcore_barrier()

    _zero_acc()
    plsc.subcore_barrier()

    def _gstage(g, ib, db):
        pltpu.async_copy(
            spsrc_hbm.at[c, s, pl.ds(g * 1024, 1024)],
            ib.at[pl.ds(0, 1024)], seme)
        pltpu.async_copy(
            spdl_hbm.at[c, s, pl.ds(g * 1024, 1024)],
            db.at[pl.ds(0, 1024)], seme)

    @pl.when(spr > 0)
    def _():
        _gstage(0, csp, cdp)

    def _grp(g, _):
        pltpu.make_async_copy(
            spsrc_hbm.at[c, s, pl.ds(g * 1024, 1024)],
            csp.at[pl.ds(0, 1024)], seme).wait()
        pltpu.make_async_copy(
            spdl_hbm.at[c, s, pl.ds(g * 1024, 1024)],
            cdp.at[pl.ds(0, 1024)], seme).wait()

        @pl.when((g & 1) == 0)
        def _even():
            @pl.when(g + 1 < spr)
            def _():
                _gstage(g + 1, csrc, cdst)
            _drain_pairs(jnp.int32(8), csp, cdp)

        @pl.when((g & 1) == 1)
        def _odd():
            @pl.when(g + 1 < spr)
            def _():
                _gstage(g + 1, csp, cdp)
            _drain_pairs(jnp.int32(8), csrc, cdst)
        return _
    lax.fori_loop(0, spr, _grp, None)

    plsc.subcore_barrier()
    pltpu.sync_copy(
        acc.at[pl.ds(s * (CHUNK // 16), CHUNK // 16), :],
        out_hbm.at[pl.ds(lo2 + s * (CHUNK // 16), CHUNK // 16), :])
    plsc.subcore_barrier()


def _stage_a0(dp_ref, deg_ref):
    deg_ref[...] = jnp.sum(dp_ref[...], axis=0)[:, None]


def _stage_a(deg_ref, x_ref, dis_ref, u16_ref):
    deg = deg_ref[...][:, 0] + 1.0
    dis = lax.rsqrt(deg)
    dis_ref[...] = dis[:, None]
    u16_ref[...] = jnp.concatenate(
        [dis[:, None] * x_ref[...], jnp.zeros((R, 14), _f32)], axis=1)


def _stage_b(t1_ref, u16_ref, dis_ref, w1_ref, b1_ref, w2_ref, u2_ref):
    t1 = t1_ref[0] + t1_ref[1]
    dis = dis_ref[...]
    s1 = dis * (t1[:, :2] + u16_ref[..., :2])
    h1 = jnp.maximum(
        jnp.dot(s1, w1_ref[...], preferred_element_type=_f32) + b1_ref[...],
        0.0)
    z = jnp.dot(h1, w2_ref[...], preferred_element_type=_f32)
    u2_ref[...] = dis * z


def _stage_c(t2_ref, u2_ref, dis_ref, b2_ref, wl_ref, bl_ref, out_ref, acc):
    i = pl.program_id(0)
    h2 = jnp.maximum(
        dis_ref[...] * (t2_ref[...] + u2_ref[...]) + b2_ref[...], 0.0)
    part = jnp.sum(h2, axis=0, keepdims=True)

    @pl.when(i == 0)
    def _():
        acc[...] = part

    @pl.when(i > 0)
    def _():
        acc[...] = acc[...] + part

    @pl.when(i == GRID - 1)
    def _():
        pooled = acc[...] / float(N_NODES)
        out_ref[...] = (
            jnp.dot(pooled, wl_ref[...], preferred_element_type=_f32)
            + bl_ref[...])


def kernel(x, edge_index, W1, b1, W2, b2, Wl, bl):
    ei = edge_index.astype(_i32)
    src = jnp.concatenate([ei[0], jnp.zeros((PAD_E,), _i32)])
    dst = jnp.concatenate([ei[1], jnp.full((PAD_E,), N_NODES, _i32)])
    src2d = src.reshape(EROWS, 128)
    dst2d = dst.reshape(EROWS, 128)

    degpart = _deg_sc(dst2d)

    deg2d = pl.pallas_call(
        _stage_a0,
        grid=(17,),
        in_specs=[pl.BlockSpec((32, 5888), lambda i: (0, i))],
        out_specs=pl.BlockSpec((5888, 1), lambda i: (i, 0)),
        out_shape=jax.ShapeDtypeStruct((N_PAD, 1), _f32),
    )(degpart)

    dis, u16 = pl.pallas_call(
        _stage_a,
        grid=(GRID,),
        in_specs=[
            pl.BlockSpec((R, 1), lambda i: (i, 0)),
            pl.BlockSpec((R, 2), lambda i: (i, 0)),
        ],
        out_specs=[
            pl.BlockSpec((R, 1), lambda i: (i, 0)),
            pl.BlockSpec((R, 16), lambda i: (i, 0)),
        ],
        out_shape=[
            jax.ShapeDtypeStruct((N_NODES, 1), _f32),
            jax.ShapeDtypeStruct((N_NODES, 16), _f32),
        ],
    )(deg2d, x)

    t1part = _seg1_sc(src2d, dst2d, u16)

    u2 = pl.pallas_call(
        _stage_b,
        grid=(GRID,),
        in_specs=[
            pl.BlockSpec((2, R, 16), lambda i: (0, i, 0)),
            pl.BlockSpec((R, 16), lambda i: (i, 0)),
            pl.BlockSpec((R, 1), lambda i: (i, 0)),
            pl.BlockSpec((2, 64), lambda i: (0, 0)),
            pl.BlockSpec((1, 64), lambda i: (0, 0)),
            pl.BlockSpec((64, 64), lambda i: (0, 0)),
        ],
        out_specs=pl.BlockSpec((R, 64), lambda i: (i, 0)),
        out_shape=jax.ShapeDtypeStruct((N_NODES, 64), _f32),
    )(t1part, u16, dis, W1, b1.reshape(1, 64), W2)

    t2, _sp1, _sp2 = _seg2_sc(src2d, dst2d, u2)

    out = pl.pallas_call(
        _stage_c,
        grid=(GRID,),
        in_specs=[
            pl.BlockSpec((R, 64), lambda i: (i, 0)),
            pl.BlockSpec((R, 64), lambda i: (i, 0)),
            pl.BlockSpec((R, 1), lambda i: (i, 0)),
            pl.BlockSpec((1, 64), lambda i: (0, 0)),
            pl.BlockSpec((64, 1), lambda i: (0, 0)),
            pl.BlockSpec((1, 1), lambda i: (0, 0)),
        ],
        out_specs=pl.BlockSpec((1, 1), lambda i: (0, 0)),
        out_shape=jax.ShapeDtypeStruct((1, 1), _f32),
        scratch_shapes=[pltpu.VMEM((1, 64), _f32)],
    )(t2, u2, dis, b2.reshape(1, 64), Wl, bl.reshape(1, 1))

    return out.reshape(1)
